# Initial kernel scaffold; baseline (speedup 1.0000x reference)
import jax
import jax.numpy as jnp
from jax import lax
from jax.experimental import pallas as pl
from jax.experimental.pallas import tpu as pltpu
from jax.experimental.pallas import tpu_sc as plsc

_N = 10000
_E = 160000
_D = 256
_H = 256
_BN_EPS = 1e-3

_NS = 16
_LANES = 16
_CHUNK = 80
_EPB = _E // _NS
_NCHUNKS = _EPB // _CHUNK
_RPS = _N // _NS
_HH = _H // 2

_BLK = 2000


def _prep_body(x_ref, w_ref, b_ref, z_ref):
    h = jnp.dot(x_ref[...], w_ref[...], preferred_element_type=jnp.float32)
    z = jax.nn.gelu(h + b_ref[...], approximate=False)
    z_ref[0] = z[:, :_HH]
    z_ref[1] = z[:, _HH:]


def _lin_body(x_ref, w_ref, p_ref):
    p_ref[...] = jnp.dot(x_ref[...], w_ref[...], preferred_element_type=jnp.float32)


def _upd_body(p_ref, s0_ref, s1_ref, cnt_ref, w0_ref, w1_ref, b_ref, o_ref):
    m = jnp.dot(s0_ref[...], w0_ref[...], preferred_element_type=jnp.float32)
    m = m + jnp.dot(s1_ref[...], w1_ref[...], preferred_element_type=jnp.float32)
    inv = 1.0 / jnp.maximum(cnt_ref[...][:, :1], 1.0)
    o_ref[...] = jax.nn.gelu(p_ref[...] + m * inv + b_ref[...], approximate=False)


def _sc_body(z2_hbm, dst_hbm, nbr_hbm, w_hbm, zr_hbm, zc_hbm,
             s0_hbm, s1_hbm, cnt_hbm,
             dst_v, nbr_v, w_v, rows_v, ones_v, acc_sh, cnt_sh, sem):
    c = lax.axis_index("c")
    s = lax.axis_index("s")
    f32 = jnp.float32

    @pl.loop(0, _CHUNK)
    def _(i):
        ones_v[i, :] = jnp.ones((_LANES,), f32)

    row0 = s * _NCHUNKS
    pltpu.sync_copy(dst_hbm.at[pl.ds(row0, _NCHUNKS)], dst_v)
    pltpu.sync_copy(nbr_hbm.at[pl.ds(row0, _NCHUNKS)], nbr_v)
    pltpu.sync_copy(w_hbm.at[pl.ds(row0, _NCHUNKS)], w_v)

    off = jnp.full((_LANES,), c * _N, jnp.int32)

    @pl.loop(0, _NCHUNKS)
    def _(j):
        for l in range(_CHUNK // _LANES):
            sl = pl.ds(l * _LANES, _LANES)
            nbr_v[j, sl] = nbr_v[j, sl] + off

    r0 = s * _RPS
    pltpu.sync_copy(zr_hbm.at[pl.ds(r0, _RPS)], acc_sh.at[pl.ds(r0, _RPS)])

    @pl.when(c == 0)
    def _():
        pltpu.sync_copy(zc_hbm.at[pl.ds(r0, _RPS)], cnt_sh.at[pl.ds(r0, _RPS)])

    plsc.subcore_barrier()

    @pl.loop(0, _NCHUNKS)
    def _(j):
        pltpu.async_copy(z2_hbm.at[nbr_v.at[j]], rows_v, sem).wait()

        @pl.loop(0, _CHUNK)
        def _(i):
            jv = jnp.full((_LANES,), j, jnp.int32)
            iv = jnp.full((_LANES,), i, jnp.int32)
            wv = plsc.load_gather(w_v, [jv, iv])
            for l in range(_HH // _LANES):
                sl = pl.ds(l * _LANES, _LANES)
                rows_v[i, sl] = rows_v[i, sl] * wv

        pltpu.sync_copy(rows_v, acc_sh.at[dst_v.at[j]], add=True)

        @pl.when(c == 0)
        def _():
            pltpu.sync_copy(ones_v, cnt_sh.at[dst_v.at[j]], add=True)

    plsc.subcore_barrier()

    @pl.when(c == 0)
    def _():
        pltpu.sync_copy(acc_sh.at[pl.ds(r0, _RPS)], s0_hbm.at[pl.ds(r0, _RPS)])
        pltpu.sync_copy(cnt_sh.at[pl.ds(r0, _RPS)], cnt_hbm.at[pl.ds(r0, _RPS)])

    @pl.when(c == 1)
    def _():
        pltpu.sync_copy(acc_sh.at[pl.ds(r0, _RPS)], s1_hbm.at[pl.ds(r0, _RPS)])


def _sc_segsum(z2, dst2, nbr2, w2, zrows, zcnt):
    f32 = jnp.float32
    mesh = plsc.VectorSubcoreMesh(core_axis_name="c", subcore_axis_name="s")
    kern = pl.kernel(
        _sc_body,
        out_type=(
            jax.ShapeDtypeStruct((_N, _HH), f32),
            jax.ShapeDtypeStruct((_N, _HH), f32),
            jax.ShapeDtypeStruct((_N, _LANES), f32),
        ),
        mesh=mesh,
        scratch_types=[
            pltpu.VMEM((_NCHUNKS, _CHUNK), jnp.int32),
            pltpu.VMEM((_NCHUNKS, _CHUNK), jnp.int32),
            pltpu.VMEM((_NCHUNKS, _CHUNK), f32),
            pltpu.VMEM((_CHUNK, _HH), f32),
            pltpu.VMEM((_CHUNK, _LANES), f32),
            pltpu.VMEM_SHARED((_N, _HH), f32),
            pltpu.VMEM_SHARED((_N, _LANES), f32),
            pltpu.SemaphoreType.DMA,
        ],
    )
    return kern(z2, dst2, nbr2, w2, zrows, zcnt)


def _tc_prepare(x, w1e, b1e):
    return pl.pallas_call(
        _prep_body,
        grid=(_N // _BLK,),
        in_specs=[
            pl.BlockSpec((_BLK, _D), lambda i: (i, 0)),
            pl.BlockSpec((_D, _H), lambda i: (0, 0)),
            pl.BlockSpec((1, _H), lambda i: (0, 0)),
        ],
        out_specs=pl.BlockSpec((2, _BLK, _HH), lambda i: (0, i, 0)),
        out_shape=jax.ShapeDtypeStruct((2, _N, _HH), jnp.float32),
    )(x, w1e, b1e)


def _tc_linear(x, w):
    return pl.pallas_call(
        _lin_body,
        grid=(_N // _BLK,),
        in_specs=[
            pl.BlockSpec((_BLK, _D), lambda i: (i, 0)),
            pl.BlockSpec((_D, _H), lambda i: (0, 0)),
        ],
        out_specs=pl.BlockSpec((_BLK, _H), lambda i: (i, 0)),
        out_shape=jax.ShapeDtypeStruct((_N, _H), jnp.float32),
    )(x, w)


def _tc_update(p, s0, s1, cnt, wb0, wb1, b2e):
    return pl.pallas_call(
        _upd_body,
        grid=(_N // _BLK,),
        in_specs=[
            pl.BlockSpec((_BLK, _H), lambda i: (i, 0)),
            pl.BlockSpec((_BLK, _HH), lambda i: (i, 0)),
            pl.BlockSpec((_BLK, _HH), lambda i: (i, 0)),
            pl.BlockSpec((_BLK, _LANES), lambda i: (i, 0)),
            pl.BlockSpec((_HH, _H), lambda i: (0, 0)),
            pl.BlockSpec((_HH, _H), lambda i: (0, 0)),
            pl.BlockSpec((1, _H), lambda i: (0, 0)),
        ],
        out_specs=pl.BlockSpec((_BLK, _H), lambda i: (i, 0)),
        out_shape=jax.ShapeDtypeStruct((_N, _H), jnp.float32),
    )(p, s0, s1, cnt, wb0, wb1, b2e)


def kernel(node_representations, edges, edge_weights, bn1_gamma, bn1_beta,
           W_prep, b_prep, bn2_gamma, bn2_beta, W_upd, b_upd):
    x = node_representations
    f32 = jnp.float32
    inv_sqrt = jnp.asarray(1.0 / jnp.sqrt(1.0 + _BN_EPS), f32)

    w1e = W_prep * (bn1_gamma * inv_sqrt)[:, None]
    b1e = (b_prep + bn1_beta @ W_prep)[None, :]
    w2e = W_upd * (bn2_gamma * inv_sqrt)[:, None]
    wt = w2e[:_D]
    wb0 = w2e[_D:_D + _HH]
    wb1 = w2e[_D + _HH:]
    b2e = (b_upd + bn2_beta @ W_upd)[None, :]

    dst2 = edges[0].reshape(_NS * _NCHUNKS, _CHUNK)
    nbr2 = edges[1].reshape(_NS * _NCHUNKS, _CHUNK)
    ew2 = edge_weights.reshape(_NS * _NCHUNKS, _CHUNK)

    z = _tc_prepare(x, w1e, b1e)
    z2 = z.reshape(2 * _N, _HH)
    p = _tc_linear(x, wt)

    zrows = jnp.zeros((_N, _HH), f32)
    zcnt = jnp.zeros((_N, _LANES), f32)
    s0, s1, cnt = _sc_segsum(z2, dst2, nbr2, ew2, zrows, zcnt)

    return _tc_update(p, s0, s1, cnt, wb0, wb1, b2e)


# speedup vs baseline: 4.5037x; 4.5037x over previous
import jax
import jax.numpy as jnp
from jax import lax
from jax.experimental import pallas as pl
from jax.experimental.pallas import tpu as pltpu
from jax.experimental.pallas import tpu_sc as plsc

_N = 10000
_E = 160000
_D = 256
_H = 256
_BN_EPS = 1e-3

_NS = 16
_LANES = 16
_CHUNK = 80
_EPB = _E // _NS
_NCHUNKS = _EPB // _CHUNK
_G = 25
_NG = _NCHUNKS // _G
_RPS = _N // _NS
_HH = _H // 2
_W144 = _HH + _LANES

_BLK = 2000


def _gelu(x):
    return 0.5 * x * (1.0 + jax.lax.erf(x * jnp.float32(0.7071067811865476)))


def _prep_body(x_ref, w_ref, b_ref, z_ref):
    h = jnp.dot(x_ref[...], w_ref[...], preferred_element_type=jnp.float32)
    z = _gelu(h + b_ref[...])
    ones = jnp.ones((z.shape[0], _LANES), jnp.float32)
    z_ref[0] = jnp.concatenate([z[:, :_HH], ones], axis=1)
    z_ref[1] = jnp.concatenate([z[:, _HH:], ones], axis=1)


def _lin_body(x_ref, w_ref, p_ref):
    p_ref[...] = jnp.dot(x_ref[...], w_ref[...], preferred_element_type=jnp.float32)


def _upd_body(p_ref, s0_ref, s1_ref, w0_ref, w1_ref, b_ref, o_ref):
    s0 = s0_ref[...]
    s1 = s1_ref[...]
    m = jnp.dot(s0[:, :_HH], w0_ref[...], preferred_element_type=jnp.float32)
    m = m + jnp.dot(s1[:, :_HH], w1_ref[...], preferred_element_type=jnp.float32)
    inv = 1.0 / jnp.maximum(s0[:, _HH:_HH + 1], 1.0)
    o_ref[...] = _gelu(p_ref[...] + m * inv + b_ref[...])


def _sc_body(z2_hbm, dst_hbm, nbr_hbm, w_hbm,
             s0_hbm, s1_hbm,
             dst_v, nbr_v, w_v, rows_v, acc_sh, sem):
    c = lax.axis_index("c")
    s = lax.axis_index("s")
    f32 = jnp.float32

    @pl.loop(0, _CHUNK)
    def _(i):
        for l in range(_W144 // _LANES):
            rows_v[i, pl.ds(l * _LANES, _LANES)] = jnp.zeros((_LANES,), f32)

    r0 = s * _RPS
    for k in range(_RPS // _CHUNK):
        pltpu.sync_copy(rows_v, acc_sh.at[pl.ds(r0 + k * _CHUNK, _CHUNK)])
    _REM = _RPS % _CHUNK
    if _REM:
        pltpu.sync_copy(
            rows_v.at[pl.ds(0, _REM)],
            acc_sh.at[pl.ds(r0 + (_RPS // _CHUNK) * _CHUNK, _REM)])

    plsc.subcore_barrier()

    off = jnp.full((_LANES,), c * _N, jnp.int32)

    @pl.loop(0, _NG)
    def _(g):
        grow = s * _NCHUNKS + g * _G
        pltpu.sync_copy(dst_hbm.at[pl.ds(grow, _G)], dst_v)
        pltpu.sync_copy(nbr_hbm.at[pl.ds(grow, _G)], nbr_v)
        pltpu.sync_copy(w_hbm.at[pl.ds(grow, _G)], w_v)

        @pl.loop(0, _G)
        def _(j):
            for l in range(_CHUNK // _LANES):
                sl = pl.ds(l * _LANES, _LANES)
                nbr_v[j, sl] = nbr_v[j, sl] + off

            pltpu.async_copy(z2_hbm.at[nbr_v.at[j]], rows_v, sem).wait()

            @pl.loop(0, _CHUNK)
            def _(i):
                jv = jnp.full((_LANES,), j, jnp.int32)
                iv = jnp.full((_LANES,), i, jnp.int32)
                wv = plsc.load_gather(w_v, [jv, iv])
                for l in range(_HH // _LANES):
                    sl = pl.ds(l * _LANES, _LANES)
                    rows_v[i, sl] = rows_v[i, sl] * wv

            pltpu.sync_copy(rows_v, acc_sh.at[dst_v.at[j]], add=True)

    plsc.subcore_barrier()

    @pl.when(c == 0)
    def _():
        pltpu.sync_copy(acc_sh.at[pl.ds(r0, _RPS)], s0_hbm.at[pl.ds(r0, _RPS)])

    @pl.when(c == 1)
    def _():
        pltpu.sync_copy(acc_sh.at[pl.ds(r0, _RPS)], s1_hbm.at[pl.ds(r0, _RPS)])


def _sc_segsum(z2, dst2, nbr2, w2):
    f32 = jnp.float32
    mesh = plsc.VectorSubcoreMesh(core_axis_name="c", subcore_axis_name="s")
    kern = pl.kernel(
        _sc_body,
        out_type=(
            jax.ShapeDtypeStruct((_N, _W144), f32),
            jax.ShapeDtypeStruct((_N, _W144), f32),
        ),
        mesh=mesh,
        scratch_types=[
            pltpu.VMEM((_G, _CHUNK), jnp.int32),
            pltpu.VMEM((_G, _CHUNK), jnp.int32),
            pltpu.VMEM((_G, _CHUNK), f32),
            pltpu.VMEM((_CHUNK, _W144), f32),
            pltpu.VMEM_SHARED((_N, _W144), f32),
            pltpu.SemaphoreType.DMA,
        ],
        compiler_params=pltpu.CompilerParams(
            use_tc_tiling_on_sc=False, needs_layout_passes=False),
    )
    return kern(z2, dst2, nbr2, w2)


def _tc_prepare(x, w1e, b1e):
    return pl.pallas_call(
        _prep_body,
        grid=(_N // _BLK,),
        in_specs=[
            pl.BlockSpec((_BLK, _D), lambda i: (i, 0)),
            pl.BlockSpec((_D, _H), lambda i: (0, 0)),
            pl.BlockSpec((1, _H), lambda i: (0, 0)),
        ],
        out_specs=pl.BlockSpec((2, _BLK, _W144), lambda i: (0, i, 0)),
        out_shape=jax.ShapeDtypeStruct((2, _N, _W144), jnp.float32),
    )(x, w1e, b1e)


def _tc_linear(x, w):
    return pl.pallas_call(
        _lin_body,
        grid=(_N // _BLK,),
        in_specs=[
            pl.BlockSpec((_BLK, _D), lambda i: (i, 0)),
            pl.BlockSpec((_D, _H), lambda i: (0, 0)),
        ],
        out_specs=pl.BlockSpec((_BLK, _H), lambda i: (i, 0)),
        out_shape=jax.ShapeDtypeStruct((_N, _H), jnp.float32),
    )(x, w)


def _tc_update(p, s0c, s1c, wb0, wb1, b2e):
    return pl.pallas_call(
        _upd_body,
        grid=(_N // _BLK,),
        in_specs=[
            pl.BlockSpec((_BLK, _H), lambda i: (i, 0)),
            pl.BlockSpec((_BLK, _W144), lambda i: (i, 0)),
            pl.BlockSpec((_BLK, _W144), lambda i: (i, 0)),
            pl.BlockSpec((_HH, _H), lambda i: (0, 0)),
            pl.BlockSpec((_HH, _H), lambda i: (0, 0)),
            pl.BlockSpec((1, _H), lambda i: (0, 0)),
        ],
        out_specs=pl.BlockSpec((_BLK, _H), lambda i: (i, 0)),
        out_shape=jax.ShapeDtypeStruct((_N, _H), jnp.float32),
    )(p, s0c, s1c, wb0, wb1, b2e)


def kernel(node_representations, edges, edge_weights, bn1_gamma, bn1_beta,
           W_prep, b_prep, bn2_gamma, bn2_beta, W_upd, b_upd):
    x = node_representations

    inv_sqrt = jnp.asarray(1.0 / jnp.sqrt(1.0 + _BN_EPS), jnp.float32)
    w1e = W_prep * (bn1_gamma * inv_sqrt)[:, None]
    b1e = (b_prep + bn1_beta @ W_prep)[None, :]
    w2e = W_upd * (bn2_gamma * inv_sqrt)[:, None]
    wt = w2e[:_D]
    wb0 = w2e[_D:_D + _HH]
    wb1 = w2e[_D + _HH:]
    b2e = (b_upd + bn2_beta @ W_upd)[None, :]

    dst2 = edges[0].reshape(_NS * _NCHUNKS, _CHUNK)
    nbr2 = edges[1].reshape(_NS * _NCHUNKS, _CHUNK)
    ew2 = edge_weights.reshape(_NS * _NCHUNKS, _CHUNK)

    z = _tc_prepare(x, w1e, b1e)
    z2 = z.reshape(2 * _N, _W144)
    p = _tc_linear(x, wt)

    s0c, s1c = _sc_segsum(z2, dst2, nbr2, ew2)

    return _tc_update(p, s0c, s1c, wb0, wb1, b2e)


# speedup vs baseline: 6.2007x; 1.3768x over previous
import jax
import jax.numpy as jnp
from jax import lax
from jax.experimental import pallas as pl
from jax.experimental.pallas import tpu as pltpu
from jax.experimental.pallas import tpu_sc as plsc

_N = 10000
_E = 160000
_D = 256
_H = 256
_BN_EPS = 1e-3

_NS = 16
_LANES = 16
_CHUNK = 80
_EPB = _E // _NS
_NCHUNKS = _EPB // _CHUNK
_G = 25
_NG = _NCHUNKS // _G
_RPS = _N // _NS
_HH = _H // 2
_W144 = _HH + _LANES

_BLK = 2000


def _gelu(x):
    return 0.5 * x * (1.0 + jax.lax.erf(x * jnp.float32(0.7071067811865476)))


def _prep_body(x_ref, w_ref, b_ref, z_ref):
    h = jnp.dot(x_ref[...], w_ref[...], preferred_element_type=jnp.float32)
    z = _gelu(h + b_ref[...])
    ones = jnp.ones((z.shape[0], _LANES), jnp.float32)
    z_ref[0] = jnp.concatenate([z[:, :_HH], ones], axis=1)
    z_ref[1] = jnp.concatenate([z[:, _HH:], ones], axis=1)


def _lin_body(x_ref, w_ref, p_ref):
    p_ref[...] = jnp.dot(x_ref[...], w_ref[...], preferred_element_type=jnp.float32)


def _upd_body(p_ref, s0_ref, s1_ref, w0_ref, w1_ref, b_ref, o_ref):
    s0 = s0_ref[...]
    s1 = s1_ref[...]
    m = jnp.dot(s0[:, :_HH], w0_ref[...], preferred_element_type=jnp.float32)
    m = m + jnp.dot(s1[:, :_HH], w1_ref[...], preferred_element_type=jnp.float32)
    inv = 1.0 / jnp.maximum(s0[:, _HH:_HH + 1], 1.0)
    o_ref[...] = _gelu(p_ref[...] + m * inv + b_ref[...])


def _sc_body(z2_hbm, dst_hbm, nbr_hbm, w_hbm,
             s0_hbm, s1_hbm,
             dst_v, nbrf_v, w_v, rows0, rows1,
             acc_sh, gsem0, gsem1, ssem0, ssem1):
    c = lax.axis_index("c")
    s = lax.axis_index("s")
    f32 = jnp.float32

    @pl.loop(0, _CHUNK)
    def _(i):
        for l in range(_W144 // _LANES):
            rows0[i, pl.ds(l * _LANES, _LANES)] = jnp.zeros((_LANES,), f32)

    r0 = s * _RPS
    for k in range(_RPS // _CHUNK):
        pltpu.sync_copy(rows0, acc_sh.at[pl.ds(r0 + k * _CHUNK, _CHUNK)])
    _REM = _RPS % _CHUNK
    if _REM:
        pltpu.sync_copy(
            rows0.at[pl.ds(0, _REM)],
            acc_sh.at[pl.ds(r0 + (_RPS // _CHUNK) * _CHUNK, _REM)])

    plsc.subcore_barrier()

    off = jnp.full((_LANES,), c * _N, jnp.int32)

    def g_desc(j, rows_b, gsem_b):
        idx = nbrf_v.at[pl.ds(j * _CHUNK, _CHUNK)]
        return pltpu.make_async_copy(z2_hbm.at[idx], rows_b, gsem_b)

    def issue_gather(j, rows_b, gsem_b):
        idx = nbrf_v.at[pl.ds(j * _CHUNK, _CHUNK)]
        pltpu.async_copy(z2_hbm.at[idx], rows_b, gsem_b)

    def sc_desc(j, rows_b, ssem_b):
        return pltpu.make_async_copy(rows_b, acc_sh.at[dst_v.at[j]], ssem_b)

    def issue_scatter(j, rows_b, ssem_b):
        pltpu.async_copy(rows_b, acc_sh.at[dst_v.at[j]], ssem_b, add=True)

    def mult(j, rows_b):
        jv = jnp.full((_LANES,), j, jnp.int32)

        @pl.loop(0, _CHUNK)
        def _(i):
            iv = jnp.full((_LANES,), i, jnp.int32)
            wv = plsc.load_gather(w_v, [jv, iv])
            for l in range(_HH // _LANES):
                sl = pl.ds(l * _LANES, _LANES)
                rows_b[i, sl] = rows_b[i, sl] * wv

    @pl.loop(0, _NG)
    def _(g):
        grow = s * _NCHUNKS + g * _G
        pltpu.sync_copy(dst_hbm.at[pl.ds(grow, _G)], dst_v)
        pltpu.sync_copy(w_hbm.at[pl.ds(grow, _G)], w_v)
        pltpu.sync_copy(
            nbr_hbm.at[pl.ds((s * _NCHUNKS + g * _G) * _CHUNK, _G * _CHUNK)],
            nbrf_v)

        @pl.loop(0, _G * _CHUNK // _LANES)
        def _(m):
            sl = pl.ds(m * _LANES, _LANES)
            nbrf_v[sl] = nbrf_v[sl] + off

        issue_gather(0, rows0, gsem0)
        issue_gather(1, rows1, gsem1)

        @pl.loop(0, (_G - 1) // 2 - 1)
        def _(r):
            j0 = 2 * r
            g_desc(j0, rows0, gsem0).wait()
            mult(j0, rows0)
            issue_scatter(j0, rows0, ssem0)
            g_desc(j0 + 1, rows1, gsem1).wait()
            mult(j0 + 1, rows1)
            issue_scatter(j0 + 1, rows1, ssem1)
            sc_desc(j0, rows0, ssem0).wait()
            issue_gather(j0 + 2, rows0, gsem0)
            sc_desc(j0 + 1, rows1, ssem1).wait()
            issue_gather(j0 + 3, rows1, gsem1)

        g_desc(_G - 3, rows0, gsem0).wait()
        mult(_G - 3, rows0)
        issue_scatter(_G - 3, rows0, ssem0)
        g_desc(_G - 2, rows1, gsem1).wait()
        mult(_G - 2, rows1)
        issue_scatter(_G - 2, rows1, ssem1)

        sc_desc(_G - 3, rows0, ssem0).wait()
        issue_gather(_G - 1, rows0, gsem0)
        g_desc(_G - 1, rows0, gsem0).wait()
        mult(_G - 1, rows0)
        pltpu.sync_copy(rows0, acc_sh.at[dst_v.at[_G - 1]], add=True)
        sc_desc(_G - 2, rows1, ssem1).wait()

    plsc.subcore_barrier()

    @pl.when(c == 0)
    def _():
        pltpu.sync_copy(acc_sh.at[pl.ds(r0, _RPS)], s0_hbm.at[pl.ds(r0, _RPS)])

    @pl.when(c == 1)
    def _():
        pltpu.sync_copy(acc_sh.at[pl.ds(r0, _RPS)], s1_hbm.at[pl.ds(r0, _RPS)])


def _sc_segsum(z2, dst2, nbr2, w2):
    f32 = jnp.float32
    mesh = plsc.VectorSubcoreMesh(core_axis_name="c", subcore_axis_name="s")
    kern = pl.kernel(
        _sc_body,
        out_type=(
            jax.ShapeDtypeStruct((_N, _W144), f32),
            jax.ShapeDtypeStruct((_N, _W144), f32),
        ),
        mesh=mesh,
        scratch_types=[
            pltpu.VMEM((_G, _CHUNK), jnp.int32),
            pltpu.VMEM((_G * _CHUNK,), jnp.int32),
            pltpu.VMEM((_G, _CHUNK), f32),
            pltpu.VMEM((_CHUNK, _W144), f32),
            pltpu.VMEM((_CHUNK, _W144), f32),
            pltpu.VMEM_SHARED((_N, _W144), f32),
            pltpu.SemaphoreType.DMA,
            pltpu.SemaphoreType.DMA,
            pltpu.SemaphoreType.DMA,
            pltpu.SemaphoreType.DMA,
        ],
        compiler_params=pltpu.CompilerParams(
            use_tc_tiling_on_sc=False, needs_layout_passes=False),
    )
    return kern(z2, dst2, nbr2, w2)


def _tc_prepare(x, w1e, b1e):
    return pl.pallas_call(
        _prep_body,
        grid=(_N // _BLK,),
        in_specs=[
            pl.BlockSpec((_BLK, _D), lambda i: (i, 0)),
            pl.BlockSpec((_D, _H), lambda i: (0, 0)),
            pl.BlockSpec((1, _H), lambda i: (0, 0)),
        ],
        out_specs=pl.BlockSpec((2, _BLK, _W144), lambda i: (0, i, 0)),
        out_shape=jax.ShapeDtypeStruct((2, _N, _W144), jnp.float32),
    )(x, w1e, b1e)


def _tc_linear(x, w):
    return pl.pallas_call(
        _lin_body,
        grid=(_N // _BLK,),
        in_specs=[
            pl.BlockSpec((_BLK, _D), lambda i: (i, 0)),
            pl.BlockSpec((_D, _H), lambda i: (0, 0)),
        ],
        out_specs=pl.BlockSpec((_BLK, _H), lambda i: (i, 0)),
        out_shape=jax.ShapeDtypeStruct((_N, _H), jnp.float32),
    )(x, w)


def _tc_update(p, s0c, s1c, wb0, wb1, b2e):
    return pl.pallas_call(
        _upd_body,
        grid=(_N // _BLK,),
        in_specs=[
            pl.BlockSpec((_BLK, _H), lambda i: (i, 0)),
            pl.BlockSpec((_BLK, _W144), lambda i: (i, 0)),
            pl.BlockSpec((_BLK, _W144), lambda i: (i, 0)),
            pl.BlockSpec((_HH, _H), lambda i: (0, 0)),
            pl.BlockSpec((_HH, _H), lambda i: (0, 0)),
            pl.BlockSpec((1, _H), lambda i: (0, 0)),
        ],
        out_specs=pl.BlockSpec((_BLK, _H), lambda i: (i, 0)),
        out_shape=jax.ShapeDtypeStruct((_N, _H), jnp.float32),
    )(p, s0c, s1c, wb0, wb1, b2e)


def kernel(node_representations, edges, edge_weights, bn1_gamma, bn1_beta,
           W_prep, b_prep, bn2_gamma, bn2_beta, W_upd, b_upd):
    x = node_representations

    inv_sqrt = jnp.asarray(1.0 / jnp.sqrt(1.0 + _BN_EPS), jnp.float32)
    w1e = W_prep * (bn1_gamma * inv_sqrt)[:, None]
    b1e = (b_prep + bn1_beta @ W_prep)[None, :]
    w2e = W_upd * (bn2_gamma * inv_sqrt)[:, None]
    wt = w2e[:_D]
    wb0 = w2e[_D:_D + _HH]
    wb1 = w2e[_D + _HH:]
    b2e = (b_upd + bn2_beta @ W_upd)[None, :]

    dst2 = edges[0].reshape(_NS * _NCHUNKS, _CHUNK)
    nbr2 = edges[1]
    ew2 = edge_weights.reshape(_NS * _NCHUNKS, _CHUNK)

    z = _tc_prepare(x, w1e, b1e)
    z2 = z.reshape(2 * _N, _W144)
    p = _tc_linear(x, wt)

    s0c, s1c = _sc_segsum(z2, dst2, nbr2, ew2)

    return _tc_update(p, s0c, s1c, wb0, wb1, b2e)


# speedup vs baseline: 7.1883x; 1.1593x over previous
import jax
import jax.numpy as jnp
from jax import lax
from jax.experimental import pallas as pl
from jax.experimental.pallas import tpu as pltpu
from jax.experimental.pallas import tpu_sc as plsc

_N = 10000
_E = 160000
_D = 256
_H = 256
_BN_EPS = 1e-3
_INV_SQRT = float(1.0 / (1.0 + _BN_EPS) ** 0.5)

_NS = 16
_LANES = 16
_CHUNK = 80
_EPB = _E // _NS
_NCHUNKS = _EPB // _CHUNK
_G = 25
_NG = _NCHUNKS // _G
_RPS = _N // _NS
_HH = _H // 2

_BLK = 2000
_NB = _N // _BLK


def _gelu(x):
    return 0.5 * x * (1.0 + jax.lax.erf(x * jnp.float32(0.7071067811865476)))


def _prep_body(x_ref, w_ref, b_ref, s_ref, beta_ref, z_ref):
    xb = x_ref[...] * (s_ref[...] * jnp.float32(_INV_SQRT)) + beta_ref[...]
    h = jnp.dot(xb, w_ref[...], preferred_element_type=jnp.float32)
    z_ref[...] = _gelu(h + b_ref[...])


def _lin_body(x_ref, w_ref, s_ref, beta_ref, p_ref):
    xb = x_ref[...] * (s_ref[...] * jnp.float32(_INV_SQRT)) + beta_ref[...]
    p_ref[...] = jnp.dot(xb, w_ref[...], preferred_element_type=jnp.float32)


def _upd_body(p_ref, s0_ref, s1_ref, c0_ref, c1_ref, w_ref, s_ref, beta_ref,
              b_ref, o_ref):
    cnt = c0_ref[...][:, :1] + c1_ref[...][:, :1]
    inv = 1.0 / jnp.maximum(cnt, 1.0)
    sca = s_ref[...] * jnp.float32(_INV_SQRT)
    a0 = s0_ref[...] * inv * sca[:, :_HH] + beta_ref[...][:, :_HH]
    a1 = s1_ref[...] * inv * sca[:, _HH:] + beta_ref[...][:, _HH:]
    w = w_ref[...]
    m = jnp.dot(a0, w[:_HH], preferred_element_type=jnp.float32)
    m = m + jnp.dot(a1, w[_HH:], preferred_element_type=jnp.float32)
    o_ref[...] = _gelu(p_ref[...] + m + b_ref[...])


def _sc_body(z2_hbm, dst_hbm, nbr_hbm, w_hbm,
             s0_hbm, s1_hbm, c0_hbm, c1_hbm,
             dst_v, nbrf_v, w_v, rows0, rows1, ones_v,
             acc_sh, cnt_sh, gsem0, gsem1, ssem0, ssem1, osem):
    c = lax.axis_index("c")
    s = lax.axis_index("s")
    f32 = jnp.float32
    r0 = s * _RPS

    @pl.loop(0, _CHUNK)
    def _(i):
        ones_v[i, :] = jnp.zeros((_LANES,), f32)

    for k in range(_RPS // _CHUNK):
        pltpu.sync_copy(ones_v, cnt_sh.at[pl.ds(r0 + k * _CHUNK, _CHUNK)])
    pltpu.sync_copy(
        ones_v.at[pl.ds(0, _RPS % _CHUNK)],
        cnt_sh.at[pl.ds(r0 + (_RPS // _CHUNK) * _CHUNK, _RPS % _CHUNK)])

    @pl.loop(0, _CHUNK)
    def _(i):
        ones_v[i, :] = jnp.ones((_LANES,), f32)

    @pl.loop(0, _CHUNK)
    def _(i):
        for l in range(_HH // _LANES):
            rows0[i, pl.ds(l * _LANES, _LANES)] = jnp.zeros((_LANES,), f32)

    for k in range(_RPS // _CHUNK):
        pltpu.sync_copy(rows0, acc_sh.at[pl.ds(r0 + k * _CHUNK, _CHUNK)])
    pltpu.sync_copy(
        rows0.at[pl.ds(0, _RPS % _CHUNK)],
        acc_sh.at[pl.ds(r0 + (_RPS // _CHUNK) * _CHUNK, _RPS % _CHUNK)])

    plsc.subcore_barrier()

    off = jnp.full((_LANES,), c * _N, jnp.int32)

    def g_desc(j, rows_b, gsem_b):
        idx = nbrf_v.at[pl.ds(j * _CHUNK, _CHUNK)]
        return pltpu.make_async_copy(z2_hbm.at[idx], rows_b, gsem_b)

    def issue_gather(j, rows_b, gsem_b):
        idx = nbrf_v.at[pl.ds(j * _CHUNK, _CHUNK)]
        pltpu.async_copy(z2_hbm.at[idx], rows_b, gsem_b)

    def sc_desc(j, rows_b, ssem_b):
        return pltpu.make_async_copy(rows_b, acc_sh.at[dst_v.at[j]], ssem_b)

    def issue_scatter(j, rows_b, ssem_b):
        pltpu.async_copy(rows_b, acc_sh.at[dst_v.at[j]], ssem_b, add=True)

    def ones_desc(j):
        return pltpu.make_async_copy(ones_v, cnt_sh.at[dst_v.at[j]], osem)

    def issue_ones(j):
        pltpu.async_copy(ones_v, cnt_sh.at[dst_v.at[j]], osem, add=True)

    def mult(j, rows_b):
        jv = jnp.full((_LANES,), j, jnp.int32)

        @pl.loop(0, _CHUNK)
        def _(i):
            iv = jnp.full((_LANES,), i, jnp.int32)
            wv = plsc.load_gather(w_v, [jv, iv])
            for l in range(_HH // _LANES):
                sl = pl.ds(l * _LANES, _LANES)
                rows_b[i, sl] = rows_b[i, sl] * wv

    def do_chunk(j, rows_b, gsem_b, ssem_b, count_par):
        g_desc(j, rows_b, gsem_b).wait()

        @pl.when(c == count_par)
        def _():
            issue_ones(j)

        mult(j, rows_b)
        issue_scatter(j, rows_b, ssem_b)

        @pl.when(c == count_par)
        def _():
            ones_desc(j).wait()

    @pl.loop(0, _NG)
    def _(g):
        grow = s * _NCHUNKS + g * _G
        pltpu.sync_copy(dst_hbm.at[pl.ds(grow, _G)], dst_v)
        pltpu.sync_copy(w_hbm.at[pl.ds(grow, _G)], w_v)
        pltpu.sync_copy(nbr_hbm.at[pl.ds(grow * _CHUNK, _G * _CHUNK)], nbrf_v)

        @pl.loop(0, _G * _CHUNK // _LANES)
        def _(m):
            sl = pl.ds(m * _LANES, _LANES)
            nbrf_v[sl] = nbrf_v[sl] + off

        issue_gather(0, rows0, gsem0)
        issue_gather(1, rows1, gsem1)

        @pl.loop(0, (_G - 1) // 2 - 1)
        def _(r):
            j0 = 2 * r
            do_chunk(j0, rows0, gsem0, ssem0, 0)
            do_chunk(j0 + 1, rows1, gsem1, ssem1, 1)
            sc_desc(j0, rows0, ssem0).wait()
            issue_gather(j0 + 2, rows0, gsem0)
            sc_desc(j0 + 1, rows1, ssem1).wait()
            issue_gather(j0 + 3, rows1, gsem1)

        do_chunk(_G - 3, rows0, gsem0, ssem0, 0)
        do_chunk(_G - 2, rows1, gsem1, ssem1, 1)

        sc_desc(_G - 3, rows0, ssem0).wait()
        issue_gather(_G - 1, rows0, gsem0)
        g_desc(_G - 1, rows0, gsem0).wait()

        @pl.when(c == 0)
        def _():
            issue_ones(_G - 1)

        mult(_G - 1, rows0)
        pltpu.sync_copy(rows0, acc_sh.at[dst_v.at[_G - 1]], add=True)

        @pl.when(c == 0)
        def _():
            ones_desc(_G - 1).wait()

        sc_desc(_G - 2, rows1, ssem1).wait()

    plsc.subcore_barrier()

    slab = pl.ds(r0, _RPS)

    @pl.when(c == 0)
    def _():
        pltpu.sync_copy(acc_sh.at[slab], s0_hbm.at[slab])
        pltpu.sync_copy(cnt_sh.at[slab], c0_hbm.at[slab])

    @pl.when(c == 1)
    def _():
        pltpu.sync_copy(acc_sh.at[slab], s1_hbm.at[slab])
        pltpu.sync_copy(cnt_sh.at[slab], c1_hbm.at[slab])


def _sc_segsum(z2, dst2, nbr1, w2):
    f32 = jnp.float32
    mesh = plsc.VectorSubcoreMesh(core_axis_name="c", subcore_axis_name="s")
    kern = pl.kernel(
        _sc_body,
        out_type=(
            jax.ShapeDtypeStruct((_N, _HH), f32),
            jax.ShapeDtypeStruct((_N, _HH), f32),
            jax.ShapeDtypeStruct((_N, _LANES), f32),
            jax.ShapeDtypeStruct((_N, _LANES), f32),
        ),
        mesh=mesh,
        scratch_types=[
            pltpu.VMEM((_G, _CHUNK), jnp.int32),
            pltpu.VMEM((_G * _CHUNK,), jnp.int32),
            pltpu.VMEM((_G, _CHUNK), f32),
            pltpu.VMEM((_CHUNK, _HH), f32),
            pltpu.VMEM((_CHUNK, _HH), f32),
            pltpu.VMEM((_CHUNK, _LANES), f32),
            pltpu.VMEM_SHARED((_N, _HH), f32),
            pltpu.VMEM_SHARED((_N, _LANES), f32),
            pltpu.SemaphoreType.DMA,
            pltpu.SemaphoreType.DMA,
            pltpu.SemaphoreType.DMA,
            pltpu.SemaphoreType.DMA,
            pltpu.SemaphoreType.DMA,
        ],
        compiler_params=pltpu.CompilerParams(
            use_tc_tiling_on_sc=False, needs_layout_passes=False),
    )
    return kern(z2, dst2, nbr1, w2)


def _tc_prepare(x, w_prep, b_prep, g1, beta1):
    return pl.pallas_call(
        _prep_body,
        grid=(2, _NB),
        in_specs=[
            pl.BlockSpec((_BLK, _D), lambda h, i: (i, 0)),
            pl.BlockSpec((_D, _HH), lambda h, i: (0, h)),
            pl.BlockSpec((1, _HH), lambda h, i: (0, h)),
            pl.BlockSpec((1, _D), lambda h, i: (0, 0)),
            pl.BlockSpec((1, _D), lambda h, i: (0, 0)),
        ],
        out_specs=pl.BlockSpec((_BLK, _HH), lambda h, i: (h * _NB + i, 0)),
        out_shape=jax.ShapeDtypeStruct((2 * _N, _HH), jnp.float32),
    )(x, w_prep, b_prep, g1, beta1)


def _tc_linear(x, w_upd, g2, beta2):
    return pl.pallas_call(
        _lin_body,
        grid=(_NB,),
        in_specs=[
            pl.BlockSpec((_BLK, _D), lambda i: (i, 0)),
            pl.BlockSpec((_D, _H), lambda i: (0, 0)),
            pl.BlockSpec((1, _D), lambda i: (0, 0)),
            pl.BlockSpec((1, _D), lambda i: (0, 0)),
        ],
        out_specs=pl.BlockSpec((_BLK, _H), lambda i: (i, 0)),
        out_shape=jax.ShapeDtypeStruct((_N, _H), jnp.float32),
    )(x, w_upd, g2, beta2)


def _tc_update(p, s0, s1, c0, c1, w_upd, g2, beta2, b_upd):
    return pl.pallas_call(
        _upd_body,
        grid=(_NB,),
        in_specs=[
            pl.BlockSpec((_BLK, _H), lambda i: (i, 0)),
            pl.BlockSpec((_BLK, _HH), lambda i: (i, 0)),
            pl.BlockSpec((_BLK, _HH), lambda i: (i, 0)),
            pl.BlockSpec((_BLK, _LANES), lambda i: (i, 0)),
            pl.BlockSpec((_BLK, _LANES), lambda i: (i, 0)),
            pl.BlockSpec((_D, _H), lambda i: (1, 0)),
            pl.BlockSpec((1, _D), lambda i: (0, 1)),
            pl.BlockSpec((1, _D), lambda i: (0, 1)),
            pl.BlockSpec((1, _H), lambda i: (0, 0)),
        ],
        out_specs=pl.BlockSpec((_BLK, _H), lambda i: (i, 0)),
        out_shape=jax.ShapeDtypeStruct((_N, _H), jnp.float32),
    )(p, s0, s1, c0, c1, w_upd, g2, beta2, b_upd)


def kernel(node_representations, edges, edge_weights, bn1_gamma, bn1_beta,
           W_prep, b_prep, bn2_gamma, bn2_beta, W_upd, b_upd):
    x = node_representations

    g1 = bn1_gamma[None, :]
    beta1 = bn1_beta[None, :]
    g2 = bn2_gamma[None, :]
    beta2 = bn2_beta[None, :]
    b_prep2 = b_prep[None, :]
    b_upd2 = b_upd[None, :]

    dst2 = edges[0].reshape(_NS * _NCHUNKS, _CHUNK)
    nbr1 = edges[1]
    ew2 = edge_weights.reshape(_NS * _NCHUNKS, _CHUNK)

    z2 = _tc_prepare(x, W_prep, b_prep2, g1, beta1)
    p = _tc_linear(x, W_upd, g2, beta2)

    s0, s1, c0, c1 = _sc_segsum(z2, dst2, nbr1, ew2)

    return _tc_update(p, s0, s1, c0, c1, W_upd, g2, beta2, b_upd2)


# speedup vs baseline: 8.4356x; 1.1735x over previous
import jax
import jax.numpy as jnp
from jax import lax
from jax.experimental import pallas as pl
from jax.experimental.pallas import tpu as pltpu
from jax.experimental.pallas import tpu_sc as plsc

_N = 10000
_E = 160000
_D = 256
_H = 256
_BN_EPS = 1e-3
_INV_SQRT = float(1.0 / (1.0 + _BN_EPS) ** 0.5)

_NS = 16
_LANES = 16
_CHUNK = 80
_EPB = _E // _NS
_NCHUNKS = _EPB // _CHUNK
_G = 25
_NG = _NCHUNKS // _G
_RPS = _N // _NS
_HH = _H // 2

_BLK = 2000
_NB = _N // _BLK


def _gelu(x):
    return 0.5 * x * (1.0 + jax.lax.erf(x * jnp.float32(0.7071067811865476)))


def _prep_body(x_ref, w_ref, b_ref, s_ref, beta_ref, z_ref):
    xb = x_ref[...] * (s_ref[...] * jnp.float32(_INV_SQRT)) + beta_ref[...]
    h = jnp.dot(xb, w_ref[...], preferred_element_type=jnp.float32)
    z_ref[...] = _gelu(h + b_ref[...])


def _lin_body(x_ref, w_ref, s_ref, beta_ref, p_ref):
    xb = x_ref[...] * (s_ref[...] * jnp.float32(_INV_SQRT)) + beta_ref[...]
    p_ref[...] = jnp.dot(xb, w_ref[...], preferred_element_type=jnp.float32)


def _upd_body(p_ref, s0_ref, s1_ref, c0_ref, c1_ref, w_ref, s_ref, beta_ref,
              b_ref, o_ref):
    cnt = c0_ref[...][:, :1] + c1_ref[...][:, :1]
    inv = 1.0 / jnp.maximum(cnt, 1.0)
    sca = s_ref[...] * jnp.float32(_INV_SQRT)
    a0 = s0_ref[...] * inv * sca[:, :_HH] + beta_ref[...][:, :_HH]
    a1 = s1_ref[...] * inv * sca[:, _HH:] + beta_ref[...][:, _HH:]
    w = w_ref[...]
    m = jnp.dot(a0, w[:_HH], preferred_element_type=jnp.float32)
    m = m + jnp.dot(a1, w[_HH:], preferred_element_type=jnp.float32)
    o_ref[...] = _gelu(p_ref[...] + m + b_ref[...])


def _sc_body(z2_hbm, dst_hbm, nbr_hbm, w_hbm,
             s0_hbm, s1_hbm, c0_hbm, c1_hbm,
             dstf_v, dst_v, nbrf_v, w_v, rows0, rows1, rows2, ones_v,
             acc_sh, cnt_sh, gsem0, gsem1, gsem2, ssem0, ssem1, ssem2, osem):
    c = lax.axis_index("c")
    s = lax.axis_index("s")
    f32 = jnp.float32
    r0 = s * _RPS

    @pl.loop(0, _CHUNK)
    def _(i):
        ones_v[i, :] = jnp.zeros((_LANES,), f32)

    for k in range(_RPS // _CHUNK):
        pltpu.sync_copy(ones_v, cnt_sh.at[pl.ds(r0 + k * _CHUNK, _CHUNK)])
    pltpu.sync_copy(
        ones_v.at[pl.ds(0, _RPS % _CHUNK)],
        cnt_sh.at[pl.ds(r0 + (_RPS // _CHUNK) * _CHUNK, _RPS % _CHUNK)])

    @pl.loop(0, _CHUNK)
    def _(i):
        ones_v[i, :] = jnp.ones((_LANES,), f32)

    @pl.loop(0, _CHUNK)
    def _(i):
        for l in range(_HH // _LANES):
            rows0[i, pl.ds(l * _LANES, _LANES)] = jnp.zeros((_LANES,), f32)

    for k in range(_RPS // _CHUNK):
        pltpu.sync_copy(rows0, acc_sh.at[pl.ds(r0 + k * _CHUNK, _CHUNK)])
    pltpu.sync_copy(
        rows0.at[pl.ds(0, _RPS % _CHUNK)],
        acc_sh.at[pl.ds(r0 + (_RPS // _CHUNK) * _CHUNK, _RPS % _CHUNK)])

    plsc.subcore_barrier()

    off = jnp.full((_LANES,), c * _N, jnp.int32)

    def g_desc(j, rows_b, gsem_b):
        idx = nbrf_v.at[pl.ds(j * _CHUNK, _CHUNK)]
        return pltpu.make_async_copy(z2_hbm.at[idx], rows_b, gsem_b)

    def issue_gather(j, rows_b, gsem_b):
        idx = nbrf_v.at[pl.ds(j * _CHUNK, _CHUNK)]
        pltpu.async_copy(z2_hbm.at[idx], rows_b, gsem_b)

    def sc_desc(j, rows_b, ssem_b):
        return pltpu.make_async_copy(rows_b, acc_sh.at[dst_v.at[j]], ssem_b)

    def issue_scatter(j, rows_b, ssem_b):
        pltpu.async_copy(rows_b, acc_sh.at[dst_v.at[j]], ssem_b, add=True)

    def ones_desc(j):
        return pltpu.make_async_copy(ones_v, cnt_sh.at[dst_v.at[j]], osem)

    def issue_ones(j):
        pltpu.async_copy(ones_v, cnt_sh.at[dst_v.at[j]], osem, add=True)

    def mult(j, rows_b):
        base = j * _CHUNK

        @pl.loop(0, _CHUNK)
        def _(i):
            iv = jnp.full((_LANES,), base + i, jnp.int32)
            wv = plsc.load_gather(w_v, [iv])
            for l in range(_HH // _LANES):
                sl = pl.ds(l * _LANES, _LANES)
                rows_b[i, sl] = rows_b[i, sl] * wv

    def do_chunk(j, rows_b, gsem_b, ssem_b):
        mine = c == lax.rem(j, 2)
        g_desc(j, rows_b, gsem_b).wait()

        @pl.when(mine)
        def _():
            issue_ones(j)

        mult(j, rows_b)
        issue_scatter(j, rows_b, ssem_b)

        @pl.when(mine)
        def _():
            ones_desc(j).wait()

    @pl.loop(0, _NG)
    def _(g):
        ebase = s * _EPB + g * _G * _CHUNK
        pltpu.sync_copy(dst_hbm.at[pl.ds(ebase, _G * _CHUNK)], dstf_v)
        pltpu.sync_copy(w_hbm.at[pl.ds(ebase, _G * _CHUNK)], w_v)
        pltpu.sync_copy(nbr_hbm.at[pl.ds(ebase, _G * _CHUNK)], nbrf_v)

        @pl.loop(0, _G * _CHUNK // _LANES)
        def _(m):
            sl = pl.ds(m * _LANES, _LANES)
            nbrf_v[sl] = nbrf_v[sl] + off

        @pl.loop(0, _G)
        def _(j):
            for l in range(_CHUNK // _LANES):
                dst_v[j, pl.ds(l * _LANES, _LANES)] = (
                    dstf_v[pl.ds(j * _CHUNK + l * _LANES, _LANES)])

        issue_gather(0, rows0, gsem0)
        issue_gather(1, rows1, gsem1)
        issue_gather(2, rows2, gsem2)

        @pl.loop(0, (_G - 1) // 3 - 1)
        def _(r):
            j0 = 3 * r
            do_chunk(j0, rows0, gsem0, ssem0)
            do_chunk(j0 + 1, rows1, gsem1, ssem1)
            sc_desc(j0, rows0, ssem0).wait()
            issue_gather(j0 + 3, rows0, gsem0)
            do_chunk(j0 + 2, rows2, gsem2, ssem2)
            sc_desc(j0 + 1, rows1, ssem1).wait()
            issue_gather(j0 + 4, rows1, gsem1)
            sc_desc(j0 + 2, rows2, ssem2).wait()
            issue_gather(j0 + 5, rows2, gsem2)

        do_chunk(_G - 4, rows0, gsem0, ssem0)
        do_chunk(_G - 3, rows1, gsem1, ssem1)
        sc_desc(_G - 4, rows0, ssem0).wait()
        issue_gather(_G - 1, rows0, gsem0)
        do_chunk(_G - 2, rows2, gsem2, ssem2)

        g_desc(_G - 1, rows0, gsem0).wait()
        lmine = c == lax.rem(_G - 1, 2)

        @pl.when(lmine)
        def _():
            issue_ones(_G - 1)

        mult(_G - 1, rows0)
        pltpu.sync_copy(rows0, acc_sh.at[dst_v.at[_G - 1]], add=True)

        @pl.when(lmine)
        def _():
            ones_desc(_G - 1).wait()

        sc_desc(_G - 3, rows1, ssem1).wait()
        sc_desc(_G - 2, rows2, ssem2).wait()

    plsc.subcore_barrier()

    slab = pl.ds(r0, _RPS)

    @pl.when(c == 0)
    def _():
        pltpu.sync_copy(acc_sh.at[slab], s0_hbm.at[slab])
        pltpu.sync_copy(cnt_sh.at[slab], c0_hbm.at[slab])

    @pl.when(c == 1)
    def _():
        pltpu.sync_copy(acc_sh.at[slab], s1_hbm.at[slab])
        pltpu.sync_copy(cnt_sh.at[slab], c1_hbm.at[slab])


def _sc_segsum(z2, dst1, nbr1, w1):
    f32 = jnp.float32
    mesh = plsc.VectorSubcoreMesh(core_axis_name="c", subcore_axis_name="s")
    kern = pl.kernel(
        _sc_body,
        out_type=(
            jax.ShapeDtypeStruct((_N, _HH), f32),
            jax.ShapeDtypeStruct((_N, _HH), f32),
            jax.ShapeDtypeStruct((_N, _LANES), f32),
            jax.ShapeDtypeStruct((_N, _LANES), f32),
        ),
        mesh=mesh,
        scratch_types=[
            pltpu.VMEM((_G * _CHUNK,), jnp.int32),
            pltpu.VMEM((_G, _CHUNK), jnp.int32),
            pltpu.VMEM((_G * _CHUNK,), jnp.int32),
            pltpu.VMEM((_G * _CHUNK,), f32),
            pltpu.VMEM((_CHUNK, _HH), f32),
            pltpu.VMEM((_CHUNK, _HH), f32),
            pltpu.VMEM((_CHUNK, _HH), f32),
            pltpu.VMEM((_CHUNK, _LANES), f32),
            pltpu.VMEM_SHARED((_N, _HH), f32),
            pltpu.VMEM_SHARED((_N, _LANES), f32),
            pltpu.SemaphoreType.DMA,
            pltpu.SemaphoreType.DMA,
            pltpu.SemaphoreType.DMA,
            pltpu.SemaphoreType.DMA,
            pltpu.SemaphoreType.DMA,
            pltpu.SemaphoreType.DMA,
            pltpu.SemaphoreType.DMA,
        ],
        compiler_params=pltpu.CompilerParams(
            use_tc_tiling_on_sc=False, needs_layout_passes=False),
    )
    return kern(z2, dst1, nbr1, w1)


def _tc_prepare(x, w_prep, b_prep, g1, beta1):
    return pl.pallas_call(
        _prep_body,
        grid=(2, _NB),
        in_specs=[
            pl.BlockSpec((_BLK, _D), lambda h, i: (i, 0)),
            pl.BlockSpec((_D, _HH), lambda h, i: (0, h)),
            pl.BlockSpec((1, _HH), lambda h, i: (0, h)),
            pl.BlockSpec((1, _D), lambda h, i: (0, 0)),
            pl.BlockSpec((1, _D), lambda h, i: (0, 0)),
        ],
        out_specs=pl.BlockSpec((_BLK, _HH), lambda h, i: (h * _NB + i, 0)),
        out_shape=jax.ShapeDtypeStruct((2 * _N, _HH), jnp.float32),
    )(x, w_prep, b_prep, g1, beta1)


def _tc_linear(x, w_upd, g2, beta2):
    return pl.pallas_call(
        _lin_body,
        grid=(_NB,),
        in_specs=[
            pl.BlockSpec((_BLK, _D), lambda i: (i, 0)),
            pl.BlockSpec((_D, _H), lambda i: (0, 0)),
            pl.BlockSpec((1, _D), lambda i: (0, 0)),
            pl.BlockSpec((1, _D), lambda i: (0, 0)),
        ],
        out_specs=pl.BlockSpec((_BLK, _H), lambda i: (i, 0)),
        out_shape=jax.ShapeDtypeStruct((_N, _H), jnp.float32),
    )(x, w_upd, g2, beta2)


def _tc_update(p, s0, s1, c0, c1, w_upd, g2, beta2, b_upd):
    return pl.pallas_call(
        _upd_body,
        grid=(_NB,),
        in_specs=[
            pl.BlockSpec((_BLK, _H), lambda i: (i, 0)),
            pl.BlockSpec((_BLK, _HH), lambda i: (i, 0)),
            pl.BlockSpec((_BLK, _HH), lambda i: (i, 0)),
            pl.BlockSpec((_BLK, _LANES), lambda i: (i, 0)),
            pl.BlockSpec((_BLK, _LANES), lambda i: (i, 0)),
            pl.BlockSpec((_D, _H), lambda i: (1, 0)),
            pl.BlockSpec((1, _D), lambda i: (0, 1)),
            pl.BlockSpec((1, _D), lambda i: (0, 1)),
            pl.BlockSpec((1, _H), lambda i: (0, 0)),
        ],
        out_specs=pl.BlockSpec((_BLK, _H), lambda i: (i, 0)),
        out_shape=jax.ShapeDtypeStruct((_N, _H), jnp.float32),
    )(p, s0, s1, c0, c1, w_upd, g2, beta2, b_upd)


def kernel(node_representations, edges, edge_weights, bn1_gamma, bn1_beta,
           W_prep, b_prep, bn2_gamma, bn2_beta, W_upd, b_upd):
    x = node_representations

    g1 = bn1_gamma[None, :]
    beta1 = bn1_beta[None, :]
    g2 = bn2_gamma[None, :]
    beta2 = bn2_beta[None, :]
    b_prep2 = b_prep[None, :]
    b_upd2 = b_upd[None, :]

    z2 = _tc_prepare(x, W_prep, b_prep2, g1, beta1)
    p = _tc_linear(x, W_upd, g2, beta2)

    s0, s1, c0, c1 = _sc_segsum(z2, edges[0], edges[1], edge_weights)

    return _tc_update(p, s0, s1, c0, c1, W_upd, g2, beta2, b_upd2)


# speedup vs baseline: 9.4127x; 1.1158x over previous
import jax
import jax.numpy as jnp
from jax import lax
from jax.experimental import pallas as pl
from jax.experimental.pallas import tpu as pltpu
from jax.experimental.pallas import tpu_sc as plsc

_N = 10000
_E = 160000
_D = 256
_H = 256
_BN_EPS = 1e-3
_INV_SQRT = float(1.0 / (1.0 + _BN_EPS) ** 0.5)

_NS = 16
_LANES = 16
_CHUNK = 80
_EPB = _E // _NS
_NCHUNKS = _EPB // _CHUNK
_G = 25
_NG = _NCHUNKS // _G
_RPS = _N // _NS
_HH = _H // 2

_BLK = 2000
_NB = _N // _BLK


def _gelu(x):
    return 0.5 * x * (1.0 + jax.lax.erf(x * jnp.float32(0.7071067811865476)))


def _prep_body(x_ref, w_ref, b_ref, s_ref, beta_ref, z_ref):
    xb = x_ref[...] * (s_ref[...] * jnp.float32(_INV_SQRT)) + beta_ref[...]
    h = jnp.dot(xb, w_ref[...], preferred_element_type=jnp.float32)
    z = _gelu(h + b_ref[...])
    z_ref[0] = z[:, :_HH]
    z_ref[1] = z[:, _HH:]


def _lin_body(x_ref, w_ref, s_ref, beta_ref, p_ref):
    xb = x_ref[...] * (s_ref[...] * jnp.float32(_INV_SQRT)) + beta_ref[...]
    p_ref[...] = jnp.dot(xb, w_ref[...], preferred_element_type=jnp.float32)


def _upd_body(p_ref, s0_ref, s1_ref, c0_ref, c1_ref, w_ref, s_ref, beta_ref,
              b_ref, o_ref):
    cnt = c0_ref[...][:, :1] + c1_ref[...][:, :1]
    inv = 1.0 / jnp.maximum(cnt, 1.0)
    sca = s_ref[...] * jnp.float32(_INV_SQRT)
    a0 = s0_ref[...] * inv * sca[:, :_HH] + beta_ref[...][:, :_HH]
    a1 = s1_ref[...] * inv * sca[:, _HH:] + beta_ref[...][:, _HH:]
    w = w_ref[...]
    m = jnp.dot(a0, w[:_HH], preferred_element_type=jnp.float32)
    m = m + jnp.dot(a1, w[_HH:], preferred_element_type=jnp.float32)
    o_ref[...] = _gelu(p_ref[...] + m + b_ref[...])


def _sc_body(z2_hbm, dst_hbm, nbr_hbm, w_hbm,
             s0_hbm, s1_hbm, c0_hbm, c1_hbm,
             dstf_v, dst_v, nbrf_v, w_v, rows0, rows1, rows2, ones_v,
             acc_sh, cnt_sh, gsem0, gsem1, gsem2, ssem0, ssem1, ssem2, osem):
    c = lax.axis_index("c")
    s = lax.axis_index("s")
    f32 = jnp.float32
    r0 = s * _RPS

    @pl.loop(0, _CHUNK)
    def _(i):
        ones_v[i, :] = jnp.zeros((_LANES,), f32)

    for k in range(_RPS // _CHUNK):
        pltpu.sync_copy(ones_v, cnt_sh.at[pl.ds(r0 + k * _CHUNK, _CHUNK)])
    pltpu.sync_copy(
        ones_v.at[pl.ds(0, _RPS % _CHUNK)],
        cnt_sh.at[pl.ds(r0 + (_RPS // _CHUNK) * _CHUNK, _RPS % _CHUNK)])

    @pl.loop(0, _CHUNK)
    def _(i):
        ones_v[i, :] = jnp.ones((_LANES,), f32)

    @pl.loop(0, _CHUNK)
    def _(i):
        for l in range(_HH // _LANES):
            rows0[i, pl.ds(l * _LANES, _LANES)] = jnp.zeros((_LANES,), f32)

    for k in range(_RPS // _CHUNK):
        pltpu.sync_copy(rows0, acc_sh.at[pl.ds(r0 + k * _CHUNK, _CHUNK)])
    pltpu.sync_copy(
        rows0.at[pl.ds(0, _RPS % _CHUNK)],
        acc_sh.at[pl.ds(r0 + (_RPS // _CHUNK) * _CHUNK, _RPS % _CHUNK)])

    plsc.subcore_barrier()

    off = jnp.full((_LANES,), c * _N, jnp.int32)

    def g_desc(j, rows_b, gsem_b):
        idx = nbrf_v.at[pl.ds(j * _CHUNK, _CHUNK)]
        return pltpu.make_async_copy(z2_hbm.at[idx], rows_b, gsem_b)

    def issue_gather(j, rows_b, gsem_b):
        idx = nbrf_v.at[pl.ds(j * _CHUNK, _CHUNK)]
        pltpu.async_copy(z2_hbm.at[idx], rows_b, gsem_b)

    def sc_desc(j, rows_b, ssem_b):
        return pltpu.make_async_copy(rows_b, acc_sh.at[dst_v.at[j]], ssem_b)

    def issue_scatter(j, rows_b, ssem_b):
        pltpu.async_copy(rows_b, acc_sh.at[dst_v.at[j]], ssem_b, add=True)

    def ones_desc(j):
        return pltpu.make_async_copy(ones_v, cnt_sh.at[dst_v.at[j]], osem)

    def issue_ones(j):
        pltpu.async_copy(ones_v, cnt_sh.at[dst_v.at[j]], osem, add=True)

    def mult(j, rows_b):
        base = j * _CHUNK

        @plsc.parallel_loop(0, _CHUNK, unroll=4)
        def _(i):
            iv = jnp.full((_LANES,), base + i, jnp.int32)
            wv = plsc.load_gather(w_v, [iv])
            for l in range(_HH // _LANES):
                sl = pl.ds(l * _LANES, _LANES)
                rows_b[i, sl] = rows_b[i, sl] * wv

    def do_chunk(j, rows_b, gsem_b, ssem_b):
        mine = c == lax.rem(j, 2)
        g_desc(j, rows_b, gsem_b).wait()

        @pl.when(mine)
        def _():
            issue_ones(j)

        mult(j, rows_b)
        issue_scatter(j, rows_b, ssem_b)

        @pl.when(mine)
        def _():
            ones_desc(j).wait()

    @pl.loop(0, _NG)
    def _(g):
        ebase = s * _EPB + g * _G * _CHUNK
        pltpu.sync_copy(dst_hbm.at[pl.ds(ebase, _G * _CHUNK)], dstf_v)
        pltpu.sync_copy(w_hbm.at[pl.ds(ebase, _G * _CHUNK)], w_v)
        pltpu.sync_copy(nbr_hbm.at[pl.ds(ebase, _G * _CHUNK)], nbrf_v)

        @pl.loop(0, _G * _CHUNK // _LANES)
        def _(m):
            sl = pl.ds(m * _LANES, _LANES)
            nbrf_v[sl] = nbrf_v[sl] + off

        @pl.loop(0, _G)
        def _(j):
            for l in range(_CHUNK // _LANES):
                dst_v[j, pl.ds(l * _LANES, _LANES)] = (
                    dstf_v[pl.ds(j * _CHUNK + l * _LANES, _LANES)])

        issue_gather(0, rows0, gsem0)
        issue_gather(1, rows1, gsem1)
        issue_gather(2, rows2, gsem2)

        @pl.loop(0, (_G - 1) // 3 - 1)
        def _(r):
            j0 = 3 * r
            do_chunk(j0, rows0, gsem0, ssem0)
            do_chunk(j0 + 1, rows1, gsem1, ssem1)
            sc_desc(j0, rows0, ssem0).wait()
            issue_gather(j0 + 3, rows0, gsem0)
            do_chunk(j0 + 2, rows2, gsem2, ssem2)
            sc_desc(j0 + 1, rows1, ssem1).wait()
            issue_gather(j0 + 4, rows1, gsem1)
            sc_desc(j0 + 2, rows2, ssem2).wait()
            issue_gather(j0 + 5, rows2, gsem2)

        do_chunk(_G - 4, rows0, gsem0, ssem0)
        do_chunk(_G - 3, rows1, gsem1, ssem1)
        sc_desc(_G - 4, rows0, ssem0).wait()
        issue_gather(_G - 1, rows0, gsem0)
        do_chunk(_G - 2, rows2, gsem2, ssem2)

        g_desc(_G - 1, rows0, gsem0).wait()
        lmine = c == lax.rem(_G - 1, 2)

        @pl.when(lmine)
        def _():
            issue_ones(_G - 1)

        mult(_G - 1, rows0)
        pltpu.sync_copy(rows0, acc_sh.at[dst_v.at[_G - 1]], add=True)

        @pl.when(lmine)
        def _():
            ones_desc(_G - 1).wait()

        sc_desc(_G - 3, rows1, ssem1).wait()
        sc_desc(_G - 2, rows2, ssem2).wait()

    plsc.subcore_barrier()

    slab = pl.ds(r0, _RPS)

    @pl.when(c == 0)
    def _():
        pltpu.sync_copy(acc_sh.at[slab], s0_hbm.at[slab])
        pltpu.sync_copy(cnt_sh.at[slab], c0_hbm.at[slab])

    @pl.when(c == 1)
    def _():
        pltpu.sync_copy(acc_sh.at[slab], s1_hbm.at[slab])
        pltpu.sync_copy(cnt_sh.at[slab], c1_hbm.at[slab])


def _sc_segsum(z2, dst1, nbr1, w1):
    f32 = jnp.float32
    mesh = plsc.VectorSubcoreMesh(core_axis_name="c", subcore_axis_name="s")
    kern = pl.kernel(
        _sc_body,
        out_type=(
            jax.ShapeDtypeStruct((_N, _HH), f32),
            jax.ShapeDtypeStruct((_N, _HH), f32),
            jax.ShapeDtypeStruct((_N, _LANES), f32),
            jax.ShapeDtypeStruct((_N, _LANES), f32),
        ),
        mesh=mesh,
        scratch_types=[
            pltpu.VMEM((_G * _CHUNK,), jnp.int32),
            pltpu.VMEM((_G, _CHUNK), jnp.int32),
            pltpu.VMEM((_G * _CHUNK,), jnp.int32),
            pltpu.VMEM((_G * _CHUNK,), f32),
            pltpu.VMEM((_CHUNK, _HH), f32),
            pltpu.VMEM((_CHUNK, _HH), f32),
            pltpu.VMEM((_CHUNK, _HH), f32),
            pltpu.VMEM((_CHUNK, _LANES), f32),
            pltpu.VMEM_SHARED((_N, _HH), f32),
            pltpu.VMEM_SHARED((_N, _LANES), f32),
            pltpu.SemaphoreType.DMA,
            pltpu.SemaphoreType.DMA,
            pltpu.SemaphoreType.DMA,
            pltpu.SemaphoreType.DMA,
            pltpu.SemaphoreType.DMA,
            pltpu.SemaphoreType.DMA,
            pltpu.SemaphoreType.DMA,
        ],
        compiler_params=pltpu.CompilerParams(
            use_tc_tiling_on_sc=False, needs_layout_passes=False),
    )
    return kern(z2, dst1, nbr1, w1)


def _tc_prepare(x, w_prep, b_prep, g1, beta1):
    z3 = pl.pallas_call(
        _prep_body,
        grid=(_NB,),
        in_specs=[
            pl.BlockSpec((_BLK, _D), lambda i: (i, 0)),
            pl.BlockSpec((_D, _H), lambda i: (0, 0)),
            pl.BlockSpec((1, _H), lambda i: (0, 0)),
            pl.BlockSpec((1, _D), lambda i: (0, 0)),
            pl.BlockSpec((1, _D), lambda i: (0, 0)),
        ],
        out_specs=pl.BlockSpec((2, _BLK, _HH), lambda i: (0, i, 0)),
        out_shape=jax.ShapeDtypeStruct((2, _N, _HH), jnp.float32),
    )(x, w_prep, b_prep, g1, beta1)
    return z3.reshape(2 * _N, _HH)


def _tc_linear(x, w_upd, g2, beta2):
    return pl.pallas_call(
        _lin_body,
        grid=(_NB,),
        in_specs=[
            pl.BlockSpec((_BLK, _D), lambda i: (i, 0)),
            pl.BlockSpec((_D, _H), lambda i: (0, 0)),
            pl.BlockSpec((1, _D), lambda i: (0, 0)),
            pl.BlockSpec((1, _D), lambda i: (0, 0)),
        ],
        out_specs=pl.BlockSpec((_BLK, _H), lambda i: (i, 0)),
        out_shape=jax.ShapeDtypeStruct((_N, _H), jnp.float32),
    )(x, w_upd, g2, beta2)


def _tc_update(p, s0, s1, c0, c1, w_upd, g2, beta2, b_upd):
    return pl.pallas_call(
        _upd_body,
        grid=(_NB,),
        in_specs=[
            pl.BlockSpec((_BLK, _H), lambda i: (i, 0)),
            pl.BlockSpec((_BLK, _HH), lambda i: (i, 0)),
            pl.BlockSpec((_BLK, _HH), lambda i: (i, 0)),
            pl.BlockSpec((_BLK, _LANES), lambda i: (i, 0)),
            pl.BlockSpec((_BLK, _LANES), lambda i: (i, 0)),
            pl.BlockSpec((_D, _H), lambda i: (1, 0)),
            pl.BlockSpec((1, _D), lambda i: (0, 1)),
            pl.BlockSpec((1, _D), lambda i: (0, 1)),
            pl.BlockSpec((1, _H), lambda i: (0, 0)),
        ],
        out_specs=pl.BlockSpec((_BLK, _H), lambda i: (i, 0)),
        out_shape=jax.ShapeDtypeStruct((_N, _H), jnp.float32),
    )(p, s0, s1, c0, c1, w_upd, g2, beta2, b_upd)


def kernel(node_representations, edges, edge_weights, bn1_gamma, bn1_beta,
           W_prep, b_prep, bn2_gamma, bn2_beta, W_upd, b_upd):
    x = node_representations

    g1 = bn1_gamma[None, :]
    beta1 = bn1_beta[None, :]
    g2 = bn2_gamma[None, :]
    beta2 = bn2_beta[None, :]
    b_prep2 = b_prep[None, :]
    b_upd2 = b_upd[None, :]

    z2 = _tc_prepare(x, W_prep, b_prep2, g1, beta1)
    p = _tc_linear(x, W_upd, g2, beta2)

    s0, s1, c0, c1 = _sc_segsum(z2, edges[0], edges[1], edge_weights)

    return _tc_update(p, s0, s1, c0, c1, W_upd, g2, beta2, b_upd2)


# speedup vs baseline: 9.6543x; 1.0257x over previous
import jax
import jax.numpy as jnp
import numpy as np
from jax import lax
from jax.experimental import pallas as pl
from jax.experimental.pallas import tpu as pltpu
from jax.experimental.pallas import tpu_sc as plsc

_N = 10000
_E = 160000
_D = 256
_H = 256
_BN_EPS = 1e-3
_INV_SQRT = float(1.0 / (1.0 + _BN_EPS) ** 0.5)

_NS = 16
_LANES = 16
_CHUNK = 80
_EPB = _E // _NS
_NCHUNKS = _EPB // _CHUNK
_G = 25
_NG = _NCHUNKS // _G
_RPS = _N // _NS
_HH = _H // 2

_BLK = 2000
_NB = _N // _BLK

_PERM = np.array(
    [32 * (p // 32) + 2 * (p % 16) + (1 if (p % 32) >= 16 else 0)
     for p in range(_HH)], dtype=np.int32)


def _gelu(x):
    return 0.5 * x * (1.0 + jax.lax.erf(x * jnp.float32(0.7071067811865476)))


def _prep_body(x_ref, w_ref, b_ref, s_ref, beta_ref, z_ref):
    xb = x_ref[...] * (s_ref[...] * jnp.float32(_INV_SQRT)) + beta_ref[...]
    h = jnp.dot(xb, w_ref[...], preferred_element_type=jnp.float32)
    z = _gelu(h + b_ref[...]).astype(jnp.bfloat16)
    z_ref[0] = z[:, :_HH]
    z_ref[1] = z[:, _HH:]


def _lin_body(x_ref, w_ref, s_ref, beta_ref, p_ref):
    xb = x_ref[...] * (s_ref[...] * jnp.float32(_INV_SQRT)) + beta_ref[...]
    p_ref[...] = jnp.dot(xb, w_ref[...], preferred_element_type=jnp.float32)


def _upd_body(p_ref, s0_ref, s1_ref, c0_ref, c1_ref, w0_ref, w1_ref, s_ref,
              beta_ref, b_ref, o_ref):
    cnt = c0_ref[...][:, :1] + c1_ref[...][:, :1]
    inv = 1.0 / jnp.maximum(cnt, 1.0)
    sca = s_ref[...] * jnp.float32(_INV_SQRT)
    a0 = s0_ref[...] * inv * sca[:, :_HH] + beta_ref[...][:, :_HH]
    a1 = s1_ref[...] * inv * sca[:, _HH:] + beta_ref[...][:, _HH:]
    m = jnp.dot(a0, w0_ref[...], preferred_element_type=jnp.float32)
    m = m + jnp.dot(a1, w1_ref[...], preferred_element_type=jnp.float32)
    o_ref[...] = _gelu(p_ref[...] + m + b_ref[...])


def _sc_body(z2_hbm, dst_hbm, nbr_hbm, w_hbm,
             s0_hbm, s1_hbm, c0_hbm, c1_hbm,
             dstf_v, dst_v, nbrf_v, w_v, gb0, gb1, fb0, fb1, ones_v,
             acc_sh, cnt_sh, gsem0, gsem1, ssem0, ssem1, osem):
    c = lax.axis_index("c")
    s = lax.axis_index("s")
    f32 = jnp.float32
    r0 = s * _RPS

    @pl.loop(0, _CHUNK)
    def _(i):
        ones_v[i, :] = jnp.zeros((_LANES,), f32)

    for k in range(_RPS // _CHUNK):
        pltpu.sync_copy(ones_v, cnt_sh.at[pl.ds(r0 + k * _CHUNK, _CHUNK)])
    pltpu.sync_copy(
        ones_v.at[pl.ds(0, _RPS % _CHUNK)],
        cnt_sh.at[pl.ds(r0 + (_RPS // _CHUNK) * _CHUNK, _RPS % _CHUNK)])

    @pl.loop(0, _CHUNK)
    def _(i):
        ones_v[i, :] = jnp.ones((_LANES,), f32)

    @pl.loop(0, _CHUNK)
    def _(i):
        for l in range(_HH // _LANES):
            fb0[i, pl.ds(l * _LANES, _LANES)] = jnp.zeros((_LANES,), f32)

    for k in range(_RPS // _CHUNK):
        pltpu.sync_copy(fb0, acc_sh.at[pl.ds(r0 + k * _CHUNK, _CHUNK)])
    pltpu.sync_copy(
        fb0.at[pl.ds(0, _RPS % _CHUNK)],
        acc_sh.at[pl.ds(r0 + (_RPS // _CHUNK) * _CHUNK, _RPS % _CHUNK)])

    plsc.subcore_barrier()

    off = jnp.full((_LANES,), c * _N, jnp.int32)

    def g_desc(j, rows_b, gsem_b):
        idx = nbrf_v.at[pl.ds(j * _CHUNK, _CHUNK)]
        return pltpu.make_async_copy(z2_hbm.at[idx], rows_b, gsem_b)

    def issue_gather(j, rows_b, gsem_b):
        idx = nbrf_v.at[pl.ds(j * _CHUNK, _CHUNK)]
        pltpu.async_copy(z2_hbm.at[idx], rows_b, gsem_b)

    def sc_desc(j, rows_b, ssem_b):
        return pltpu.make_async_copy(rows_b, acc_sh.at[dst_v.at[j]], ssem_b)

    def issue_scatter(j, rows_b, ssem_b):
        pltpu.async_copy(rows_b, acc_sh.at[dst_v.at[j]], ssem_b, add=True)

    def ones_desc(j):
        return pltpu.make_async_copy(ones_v, cnt_sh.at[dst_v.at[j]], osem)

    def issue_ones(j):
        pltpu.async_copy(ones_v, cnt_sh.at[dst_v.at[j]], osem, add=True)

    def mult(j, gb, fb):
        base = j * _CHUNK

        @plsc.parallel_loop(0, _CHUNK, unroll=4)
        def _(i):
            iv = jnp.full((_LANES,), base + i, jnp.int32)
            wv = plsc.load_gather(w_v, [iv])
            for q in range(_HH // 32):
                x32 = gb[i, pl.ds(32 * q, 32)]
                xi = plsc.bitcast(x32, jnp.int32)
                ev = plsc.bitcast(xi << 16, f32)
                od = plsc.bitcast(xi & jnp.int32(-65536), f32)
                fb[i, pl.ds(32 * q, _LANES)] = ev * wv
                fb[i, pl.ds(32 * q + _LANES, _LANES)] = od * wv

    def do_chunk(j, gb, fb, gsem_b, ssem_b):
        mine = c == lax.rem(j, 2)
        g_desc(j, gb, gsem_b).wait()

        @pl.when(mine)
        def _():
            issue_ones(j)

        @pl.when(j >= 2)
        def _():
            sc_desc(j - 2, fb, ssem_b).wait()

        mult(j, gb, fb)
        issue_scatter(j, fb, ssem_b)

        @pl.when(mine)
        def _():
            ones_desc(j).wait()

    @pl.loop(0, _NG)
    def _(g):
        ebase = s * _EPB + g * _G * _CHUNK
        pltpu.sync_copy(dst_hbm.at[pl.ds(ebase, _G * _CHUNK)], dstf_v)
        pltpu.sync_copy(w_hbm.at[pl.ds(ebase, _G * _CHUNK)], w_v)
        pltpu.sync_copy(nbr_hbm.at[pl.ds(ebase, _G * _CHUNK)], nbrf_v)

        @pl.loop(0, _G * _CHUNK // _LANES)
        def _(m):
            sl = pl.ds(m * _LANES, _LANES)
            nbrf_v[sl] = nbrf_v[sl] + off

        @pl.loop(0, _G)
        def _(j):
            for l in range(_CHUNK // _LANES):
                dst_v[j, pl.ds(l * _LANES, _LANES)] = (
                    dstf_v[pl.ds(j * _CHUNK + l * _LANES, _LANES)])

        issue_gather(0, gb0, gsem0)
        issue_gather(1, gb1, gsem1)

        @pl.loop(0, (_G - 1) // 2 - 1)
        def _(r):
            j0 = 2 * r
            do_chunk(j0, gb0, fb0, gsem0, ssem0)
            issue_gather(j0 + 2, gb0, gsem0)
            do_chunk(j0 + 1, gb1, fb1, gsem1, ssem1)
            issue_gather(j0 + 3, gb1, gsem1)

        do_chunk(_G - 3, gb0, fb0, gsem0, ssem0)
        issue_gather(_G - 1, gb0, gsem0)
        do_chunk(_G - 2, gb1, fb1, gsem1, ssem1)

        g_desc(_G - 1, gb0, gsem0).wait()
        lmine = c == lax.rem(_G - 1, 2)

        @pl.when(lmine)
        def _():
            issue_ones(_G - 1)

        sc_desc(_G - 3, fb0, ssem0).wait()
        mult(_G - 1, gb0, fb0)
        pltpu.sync_copy(fb0, acc_sh.at[dst_v.at[_G - 1]], add=True)

        @pl.when(lmine)
        def _():
            ones_desc(_G - 1).wait()

        sc_desc(_G - 2, fb1, ssem1).wait()

    plsc.subcore_barrier()

    slab = pl.ds(r0, _RPS)

    @pl.when(c == 0)
    def _():
        pltpu.sync_copy(acc_sh.at[slab], s0_hbm.at[slab])
        pltpu.sync_copy(cnt_sh.at[slab], c0_hbm.at[slab])

    @pl.when(c == 1)
    def _():
        pltpu.sync_copy(acc_sh.at[slab], s1_hbm.at[slab])
        pltpu.sync_copy(cnt_sh.at[slab], c1_hbm.at[slab])


def _sc_segsum(z2, dst1, nbr1, w1):
    f32 = jnp.float32
    mesh = plsc.VectorSubcoreMesh(core_axis_name="c", subcore_axis_name="s")
    kern = pl.kernel(
        _sc_body,
        out_type=(
            jax.ShapeDtypeStruct((_N, _HH), f32),
            jax.ShapeDtypeStruct((_N, _HH), f32),
            jax.ShapeDtypeStruct((_N, _LANES), f32),
            jax.ShapeDtypeStruct((_N, _LANES), f32),
        ),
        mesh=mesh,
        scratch_types=[
            pltpu.VMEM((_G * _CHUNK,), jnp.int32),
            pltpu.VMEM((_G, _CHUNK), jnp.int32),
            pltpu.VMEM((_G * _CHUNK,), jnp.int32),
            pltpu.VMEM((_G * _CHUNK,), f32),
            pltpu.VMEM((_CHUNK, _HH), jnp.bfloat16),
            pltpu.VMEM((_CHUNK, _HH), jnp.bfloat16),
            pltpu.VMEM((_CHUNK, _HH), f32),
            pltpu.VMEM((_CHUNK, _HH), f32),
            pltpu.VMEM((_CHUNK, _LANES), f32),
            pltpu.VMEM_SHARED((_N, _HH), f32),
            pltpu.VMEM_SHARED((_N, _LANES), f32),
            pltpu.SemaphoreType.DMA,
            pltpu.SemaphoreType.DMA,
            pltpu.SemaphoreType.DMA,
            pltpu.SemaphoreType.DMA,
            pltpu.SemaphoreType.DMA,
        ],
        compiler_params=pltpu.CompilerParams(
            use_tc_tiling_on_sc=False, needs_layout_passes=False),
    )
    return kern(z2, dst1, nbr1, w1)


def _tc_prepare(x, w_prep, b_prep, g1, beta1):
    z3 = pl.pallas_call(
        _prep_body,
        grid=(_NB,),
        in_specs=[
            pl.BlockSpec((_BLK, _D), lambda i: (i, 0)),
            pl.BlockSpec((_D, _H), lambda i: (0, 0)),
            pl.BlockSpec((1, _H), lambda i: (0, 0)),
            pl.BlockSpec((1, _D), lambda i: (0, 0)),
            pl.BlockSpec((1, _D), lambda i: (0, 0)),
        ],
        out_specs=pl.BlockSpec((2, _BLK, _HH), lambda i: (0, i, 0)),
        out_shape=jax.ShapeDtypeStruct((2, _N, _HH), jnp.bfloat16),
    )(x, w_prep, b_prep, g1, beta1)
    return z3.reshape(2 * _N, _HH)


def _tc_linear(x, w_upd, g2, beta2):
    return pl.pallas_call(
        _lin_body,
        grid=(_NB,),
        in_specs=[
            pl.BlockSpec((_BLK, _D), lambda i: (i, 0)),
            pl.BlockSpec((_D, _H), lambda i: (0, 0)),
            pl.BlockSpec((1, _D), lambda i: (0, 0)),
            pl.BlockSpec((1, _D), lambda i: (0, 0)),
        ],
        out_specs=pl.BlockSpec((_BLK, _H), lambda i: (i, 0)),
        out_shape=jax.ShapeDtypeStruct((_N, _H), jnp.float32),
    )(x, w_upd, g2, beta2)


def _tc_update(p, s0, s1, c0, c1, wb0p, wb1p, g2p, beta2p, b_upd):
    return pl.pallas_call(
        _upd_body,
        grid=(_NB,),
        in_specs=[
            pl.BlockSpec((_BLK, _H), lambda i: (i, 0)),
            pl.BlockSpec((_BLK, _HH), lambda i: (i, 0)),
            pl.BlockSpec((_BLK, _HH), lambda i: (i, 0)),
            pl.BlockSpec((_BLK, _LANES), lambda i: (i, 0)),
            pl.BlockSpec((_BLK, _LANES), lambda i: (i, 0)),
            pl.BlockSpec((_HH, _H), lambda i: (0, 0)),
            pl.BlockSpec((_HH, _H), lambda i: (0, 0)),
            pl.BlockSpec((1, _D), lambda i: (0, 0)),
            pl.BlockSpec((1, _D), lambda i: (0, 0)),
            pl.BlockSpec((1, _H), lambda i: (0, 0)),
        ],
        out_specs=pl.BlockSpec((_BLK, _H), lambda i: (i, 0)),
        out_shape=jax.ShapeDtypeStruct((_N, _H), jnp.float32),
    )(p, s0, s1, c0, c1, wb0p, wb1p, g2p, beta2p, b_upd)


def kernel(node_representations, edges, edge_weights, bn1_gamma, bn1_beta,
           W_prep, b_prep, bn2_gamma, bn2_beta, W_upd, b_upd):
    x = node_representations

    g1 = bn1_gamma[None, :]
    beta1 = bn1_beta[None, :]
    g2 = bn2_gamma[None, :]
    beta2 = bn2_beta[None, :]
    b_prep2 = b_prep[None, :]
    b_upd2 = b_upd[None, :]

    perm = jnp.asarray(_PERM)
    wb0p = W_upd[_D:_D + _HH][perm]
    wb1p = W_upd[_D + _HH:][perm]
    g2p = jnp.concatenate(
        [bn2_gamma[_D:_D + _HH][perm], bn2_gamma[_D + _HH:][perm]])[None, :]
    beta2p = jnp.concatenate(
        [bn2_beta[_D:_D + _HH][perm], bn2_beta[_D + _HH:][perm]])[None, :]

    z2 = _tc_prepare(x, W_prep, b_prep2, g1, beta1)
    p = _tc_linear(x, W_upd, g2, beta2)

    s0, s1, c0, c1 = _sc_segsum(z2, edges[0], edges[1], edge_weights)

    return _tc_update(p, s0, s1, c0, c1, wb0p, wb1p, g2p, beta2p, b_upd2)


# speedup vs baseline: 10.0157x; 1.0374x over previous
import jax
import jax.numpy as jnp
import numpy as np
from jax import lax
from jax.experimental import pallas as pl
from jax.experimental.pallas import tpu as pltpu
from jax.experimental.pallas import tpu_sc as plsc

_N = 10000
_E = 160000
_D = 256
_H = 256
_BN_EPS = 1e-3
_INV_SQRT = float(1.0 / (1.0 + _BN_EPS) ** 0.5)

_NS = 16
_LANES = 16
_CHUNK = 80
_EPB = _E // _NS
_NCHUNKS = _EPB // _CHUNK
_G = 25
_NG = _NCHUNKS // _G
_RPS = _N // _NS
_HH = _H // 2

_BLK = 2000
_NB = _N // _BLK

_PERM = np.array(
    [32 * (p // 32) + 2 * (p % 16) + (1 if (p % 32) >= 16 else 0)
     for p in range(_HH)], dtype=np.int32)


def _gelu(x):
    return 0.5 * x * (1.0 + jax.lax.erf(x * jnp.float32(0.7071067811865476)))


def _prep_body(x_ref, w_ref, b_ref, s_ref, beta_ref, e_ref,
               z_ref, dst_ref, nbr_ref):
    xb = x_ref[...] * (s_ref[...] * jnp.float32(_INV_SQRT)) + beta_ref[...]
    h = jnp.dot(xb, w_ref[...], preferred_element_type=jnp.float32)
    z = _gelu(h + b_ref[...]).astype(jnp.bfloat16)
    z_ref[0] = z[:, :_HH]
    z_ref[1] = z[:, _HH:]
    @pl.when(pl.program_id(0) == 0)
    def _():
        dst_ref[...] = e_ref[0]
        nbr_ref[...] = e_ref[1]


def _lin_body(x_ref, w_ref, s_ref, beta_ref, p_ref):
    xb = x_ref[...] * (s_ref[...] * jnp.float32(_INV_SQRT)) + beta_ref[...]
    p_ref[...] = jnp.dot(xb, w_ref[...], preferred_element_type=jnp.float32)


def _upd_body(p_ref, s0_ref, s1_ref, c0_ref, c1_ref, w0_ref, w1_ref, s_ref,
              beta_ref, b_ref, o_ref):
    cnt = c0_ref[...][:, :1] + c1_ref[...][:, :1]
    inv = 1.0 / jnp.maximum(cnt, 1.0)
    sca = s_ref[...] * jnp.float32(_INV_SQRT)
    a0 = s0_ref[...] * inv * sca[:, :_HH] + beta_ref[...][:, :_HH]
    a1 = s1_ref[...] * inv * sca[:, _HH:] + beta_ref[...][:, _HH:]
    m = jnp.dot(a0, w0_ref[...], preferred_element_type=jnp.float32)
    m = m + jnp.dot(a1, w1_ref[...], preferred_element_type=jnp.float32)
    o_ref[...] = _gelu(p_ref[...] + m + b_ref[...])


def _sc_body(z2_hbm, dst_hbm, nbr_hbm, w_hbm,
             s0_hbm, s1_hbm, c0_hbm, c1_hbm,
             dstf_v, dst_v, nbrf_v, w_v, gb0, gb1, fb0, fb1, ones_v,
             acc_sh, cnt_sh, gsem0, gsem1, ssem0, ssem1, osem):
    c = lax.axis_index("c")
    s = lax.axis_index("s")
    f32 = jnp.float32
    r0 = s * _RPS

    @pl.loop(0, _CHUNK)
    def _(i):
        ones_v[i, :] = jnp.zeros((_LANES,), f32)

    for k in range(_RPS // _CHUNK):
        pltpu.sync_copy(ones_v, cnt_sh.at[pl.ds(r0 + k * _CHUNK, _CHUNK)])
    pltpu.sync_copy(
        ones_v.at[pl.ds(0, _RPS % _CHUNK)],
        cnt_sh.at[pl.ds(r0 + (_RPS // _CHUNK) * _CHUNK, _RPS % _CHUNK)])

    @pl.loop(0, _CHUNK)
    def _(i):
        ones_v[i, :] = jnp.ones((_LANES,), f32)

    @pl.loop(0, _CHUNK)
    def _(i):
        for l in range(_HH // _LANES):
            fb0[i, pl.ds(l * _LANES, _LANES)] = jnp.zeros((_LANES,), f32)

    for k in range(_RPS // _CHUNK):
        pltpu.sync_copy(fb0, acc_sh.at[pl.ds(r0 + k * _CHUNK, _CHUNK)])
    pltpu.sync_copy(
        fb0.at[pl.ds(0, _RPS % _CHUNK)],
        acc_sh.at[pl.ds(r0 + (_RPS // _CHUNK) * _CHUNK, _RPS % _CHUNK)])

    plsc.subcore_barrier()

    off = jnp.full((_LANES,), c * _N, jnp.int32)

    def g_desc(j, rows_b, gsem_b):
        idx = nbrf_v.at[pl.ds(j * _CHUNK, _CHUNK)]
        return pltpu.make_async_copy(z2_hbm.at[idx], rows_b, gsem_b)

    def issue_gather(j, rows_b, gsem_b):
        idx = nbrf_v.at[pl.ds(j * _CHUNK, _CHUNK)]
        pltpu.async_copy(z2_hbm.at[idx], rows_b, gsem_b)

    def sc_desc(j, rows_b, ssem_b):
        return pltpu.make_async_copy(rows_b, acc_sh.at[dst_v.at[j]], ssem_b)

    def issue_scatter(j, rows_b, ssem_b):
        pltpu.async_copy(rows_b, acc_sh.at[dst_v.at[j]], ssem_b, add=True)

    def ones_desc(j):
        return pltpu.make_async_copy(ones_v, cnt_sh.at[dst_v.at[j]], osem)

    def issue_ones(j):
        pltpu.async_copy(ones_v, cnt_sh.at[dst_v.at[j]], osem, add=True)

    def mult(j, gb, fb):
        base = j * _CHUNK

        @plsc.parallel_loop(0, _CHUNK, unroll=8)
        def _(i):
            iv = jnp.full((_LANES,), base + i, jnp.int32)
            wv = plsc.load_gather(w_v, [iv])
            for q in range(_HH // 32):
                x32 = gb[i, pl.ds(32 * q, 32)]
                xi = plsc.bitcast(x32, jnp.int32)
                ev = plsc.bitcast(xi << 16, f32)
                od = plsc.bitcast(xi & jnp.int32(-65536), f32)
                fb[i, pl.ds(32 * q, _LANES)] = ev * wv
                fb[i, pl.ds(32 * q + _LANES, _LANES)] = od * wv

    def do_chunk(j, gb, fb, gsem_b, ssem_b):
        mine = c == lax.rem(j, 2)
        g_desc(j, gb, gsem_b).wait()

        @pl.when(mine)
        def _():
            issue_ones(j)

        @pl.when(j >= 2)
        def _():
            sc_desc(j - 2, fb, ssem_b).wait()

        mult(j, gb, fb)
        issue_scatter(j, fb, ssem_b)

        @pl.when(mine)
        def _():
            ones_desc(j).wait()

    @pl.loop(0, _NG)
    def _(g):
        ebase = s * _EPB + g * _G * _CHUNK
        pltpu.sync_copy(dst_hbm.at[pl.ds(ebase, _G * _CHUNK)], dstf_v)
        pltpu.sync_copy(w_hbm.at[pl.ds(ebase, _G * _CHUNK)], w_v)
        pltpu.sync_copy(nbr_hbm.at[pl.ds(ebase, _G * _CHUNK)], nbrf_v)

        @pl.loop(0, _G * _CHUNK // _LANES)
        def _(m):
            sl = pl.ds(m * _LANES, _LANES)
            nbrf_v[sl] = nbrf_v[sl] + off

        @pl.loop(0, _G)
        def _(j):
            for l in range(_CHUNK // _LANES):
                dst_v[j, pl.ds(l * _LANES, _LANES)] = (
                    dstf_v[pl.ds(j * _CHUNK + l * _LANES, _LANES)])

        issue_gather(0, gb0, gsem0)
        issue_gather(1, gb1, gsem1)

        @pl.loop(0, (_G - 1) // 2 - 1)
        def _(r):
            j0 = 2 * r
            do_chunk(j0, gb0, fb0, gsem0, ssem0)
            issue_gather(j0 + 2, gb0, gsem0)
            do_chunk(j0 + 1, gb1, fb1, gsem1, ssem1)
            issue_gather(j0 + 3, gb1, gsem1)

        do_chunk(_G - 3, gb0, fb0, gsem0, ssem0)
        issue_gather(_G - 1, gb0, gsem0)
        do_chunk(_G - 2, gb1, fb1, gsem1, ssem1)

        g_desc(_G - 1, gb0, gsem0).wait()
        lmine = c == lax.rem(_G - 1, 2)

        @pl.when(lmine)
        def _():
            issue_ones(_G - 1)

        sc_desc(_G - 3, fb0, ssem0).wait()
        mult(_G - 1, gb0, fb0)
        pltpu.sync_copy(fb0, acc_sh.at[dst_v.at[_G - 1]], add=True)

        @pl.when(lmine)
        def _():
            ones_desc(_G - 1).wait()

        sc_desc(_G - 2, fb1, ssem1).wait()

    plsc.subcore_barrier()

    slab = pl.ds(r0, _RPS)

    @pl.when(c == 0)
    def _():
        pltpu.sync_copy(acc_sh.at[slab], s0_hbm.at[slab])
        pltpu.sync_copy(cnt_sh.at[slab], c0_hbm.at[slab])

    @pl.when(c == 1)
    def _():
        pltpu.sync_copy(acc_sh.at[slab], s1_hbm.at[slab])
        pltpu.sync_copy(cnt_sh.at[slab], c1_hbm.at[slab])


def _sc_segsum(z2, dst1, nbr1, w1):
    f32 = jnp.float32
    mesh = plsc.VectorSubcoreMesh(core_axis_name="c", subcore_axis_name="s")
    kern = pl.kernel(
        _sc_body,
        out_type=(
            jax.ShapeDtypeStruct((_N, _HH), f32),
            jax.ShapeDtypeStruct((_N, _HH), f32),
            jax.ShapeDtypeStruct((_N, _LANES), f32),
            jax.ShapeDtypeStruct((_N, _LANES), f32),
        ),
        mesh=mesh,
        scratch_types=[
            pltpu.VMEM((_G * _CHUNK,), jnp.int32),
            pltpu.VMEM((_G, _CHUNK), jnp.int32),
            pltpu.VMEM((_G * _CHUNK,), jnp.int32),
            pltpu.VMEM((_G * _CHUNK,), f32),
            pltpu.VMEM((_CHUNK, _HH), jnp.bfloat16),
            pltpu.VMEM((_CHUNK, _HH), jnp.bfloat16),
            pltpu.VMEM((_CHUNK, _HH), f32),
            pltpu.VMEM((_CHUNK, _HH), f32),
            pltpu.VMEM((_CHUNK, _LANES), f32),
            pltpu.VMEM_SHARED((_N, _HH), f32),
            pltpu.VMEM_SHARED((_N, _LANES), f32),
            pltpu.SemaphoreType.DMA,
            pltpu.SemaphoreType.DMA,
            pltpu.SemaphoreType.DMA,
            pltpu.SemaphoreType.DMA,
            pltpu.SemaphoreType.DMA,
        ],
        compiler_params=pltpu.CompilerParams(
            use_tc_tiling_on_sc=False, needs_layout_passes=False),
    )
    return kern(z2, dst1, nbr1, w1)


def _tc_prepare(x, w_prep, b_prep, g1, beta1, edges):
    _EB = _E // _NB
    z3, dst1, nbr1 = pl.pallas_call(
        _prep_body,
        grid=(_NB,),
        in_specs=[
            pl.BlockSpec((_BLK, _D), lambda i: (i, 0)),
            pl.BlockSpec((_D, _H), lambda i: (0, 0)),
            pl.BlockSpec((1, _H), lambda i: (0, 0)),
            pl.BlockSpec((1, _D), lambda i: (0, 0)),
            pl.BlockSpec((1, _D), lambda i: (0, 0)),
            pl.BlockSpec((2, _E), lambda i: (0, 0)),
        ],
        out_specs=[
            pl.BlockSpec((2, _BLK, _HH), lambda i: (0, i, 0)),
            pl.BlockSpec((_E,), lambda i: (0,)),
            pl.BlockSpec((_E,), lambda i: (0,)),
        ],
        out_shape=[
            jax.ShapeDtypeStruct((2, _N, _HH), jnp.bfloat16),
            jax.ShapeDtypeStruct((_E,), jnp.int32),
            jax.ShapeDtypeStruct((_E,), jnp.int32),
        ],
    )(x, w_prep, b_prep, g1, beta1, edges)
    return z3.reshape(2 * _N, _HH), dst1, nbr1


def _tc_linear(x, w_upd, g2, beta2):
    return pl.pallas_call(
        _lin_body,
        grid=(_NB,),
        in_specs=[
            pl.BlockSpec((_BLK, _D), lambda i: (i, 0)),
            pl.BlockSpec((_D, _H), lambda i: (0, 0)),
            pl.BlockSpec((1, _D), lambda i: (0, 0)),
            pl.BlockSpec((1, _D), lambda i: (0, 0)),
        ],
        out_specs=pl.BlockSpec((_BLK, _H), lambda i: (i, 0)),
        out_shape=jax.ShapeDtypeStruct((_N, _H), jnp.float32),
    )(x, w_upd, g2, beta2)


def _tc_update(p, s0, s1, c0, c1, wb0p, wb1p, g2p, beta2p, b_upd):
    return pl.pallas_call(
        _upd_body,
        grid=(_NB,),
        in_specs=[
            pl.BlockSpec((_BLK, _H), lambda i: (i, 0)),
            pl.BlockSpec((_BLK, _HH), lambda i: (i, 0)),
            pl.BlockSpec((_BLK, _HH), lambda i: (i, 0)),
            pl.BlockSpec((_BLK, _LANES), lambda i: (i, 0)),
            pl.BlockSpec((_BLK, _LANES), lambda i: (i, 0)),
            pl.BlockSpec((_HH, _H), lambda i: (0, 0)),
            pl.BlockSpec((_HH, _H), lambda i: (0, 0)),
            pl.BlockSpec((1, _D), lambda i: (0, 0)),
            pl.BlockSpec((1, _D), lambda i: (0, 0)),
            pl.BlockSpec((1, _H), lambda i: (0, 0)),
        ],
        out_specs=pl.BlockSpec((_BLK, _H), lambda i: (i, 0)),
        out_shape=jax.ShapeDtypeStruct((_N, _H), jnp.float32),
    )(p, s0, s1, c0, c1, wb0p, wb1p, g2p, beta2p, b_upd)


def kernel(node_representations, edges, edge_weights, bn1_gamma, bn1_beta,
           W_prep, b_prep, bn2_gamma, bn2_beta, W_upd, b_upd):
    x = node_representations

    g1 = bn1_gamma[None, :]
    beta1 = bn1_beta[None, :]
    g2 = bn2_gamma[None, :]
    beta2 = bn2_beta[None, :]
    b_prep2 = b_prep[None, :]
    b_upd2 = b_upd[None, :]

    perm = jnp.asarray(_PERM)
    wb0p = W_upd[_D:_D + _HH][perm]
    wb1p = W_upd[_D + _HH:][perm]
    g2p = jnp.concatenate(
        [bn2_gamma[_D:_D + _HH][perm], bn2_gamma[_D + _HH:][perm]])[None, :]
    beta2p = jnp.concatenate(
        [bn2_beta[_D:_D + _HH][perm], bn2_beta[_D + _HH:][perm]])[None, :]

    z2, dst1, nbr1 = _tc_prepare(x, W_prep, b_prep2, g1, beta1, edges)
    p = _tc_linear(x, W_upd, g2, beta2)

    s0, s1, c0, c1 = _sc_segsum(z2, dst1, nbr1, edge_weights)

    return _tc_update(p, s0, s1, c0, c1, wb0p, wb1p, g2p, beta2p, b_upd2)
